# baseline (speedup 1.0000x reference)
import functools

import jax
import jax.numpy as jnp
from jax import lax
from jax.experimental import pallas as pl
from jax.experimental.pallas import tpu as pltpu
from jax.experimental.pallas import tpu_sc as plsc

_IN_CH = 128
_HID = 128
_OUT = 64
_N_SRC = 100000
_ND0 = 20000
_ND1 = 4096
_E0 = 320000
_E1 = 65536
_NC = 2
_NS = 16

_CH = 64
_NRING = 4

_HALF0 = _ND0 // 2
_ROWS0 = 10240
_GARB0 = _HALF0
_EPT0 = 20480
_E0P = _EPT0 * _NS
_NCH0 = _EPT0 // _CH

_EPC1 = _E1 // _NC
_EPT1 = _EPC1 // _NS
_NCH1 = _EPT1 // _CH

_CBLK = 2000
_BBLK = 1000
_HBLK = 400
_FBLK = 512


def _colsum_body(x_ref, out_ref):
    i = pl.program_id(0)

    @pl.when(i == 0)
    def _():
        out_ref[...] = jnp.zeros_like(out_ref)

    ssum = jnp.sum(x_ref[...], axis=0, keepdims=True)
    out_ref[0:1, :] += ssum

    @pl.when(i < _ND0 // _CBLK)
    def _():
        out_ref[1:2, :] += ssum


def _binarize_body(sums_ref, x_ref, xb_ref, xtb_ref, *, nfull, npart):
    m_full = sums_ref[0:1, :] / nfull
    m_part = sums_ref[1:2, :] / npart
    blk = x_ref[...]
    xb_ref[...] = jnp.sign(blk - m_full)
    xtb_ref[...] = jnp.sign(blk - m_part)


def _layer0_body(agg_ref, cnt_ref, xtb_ref, wl_ref, wr_ref, b_ref,
                 h_ref, hsums_ref):
    i = pl.program_id(0)

    @pl.when(i == 0)
    def _():
        hsums_ref[...] = jnp.zeros_like(hsums_ref)

    cnt = jnp.maximum(cnt_ref[...], 1.0)
    ma = agg_ref[...] / cnt
    hblk = (jnp.dot(ma, wl_ref[...], preferred_element_type=jnp.float32)
            + jnp.dot(xtb_ref[...], wr_ref[...], preferred_element_type=jnp.float32)
            + b_ref[...])
    hblk = jnp.maximum(hblk, 0.0)
    h_ref[...] = hblk
    hsums_ref[0:1, :] += jnp.sum(hblk, axis=0, keepdims=True)
    rows = i * _HBLK + lax.broadcasted_iota(jnp.int32, (_HBLK, 1), 0)
    hsums_ref[1:2, :] += jnp.sum(jnp.where(rows < _ND1, hblk, 0.0),
                                 axis=0, keepdims=True)


def _final_body(aggA_ref, aggB_ref, cntA_ref, cntB_ref, htb_ref,
                wl_ref, wr_ref, b_ref, out_ref):
    cnt = jnp.maximum(cntA_ref[...] + cntB_ref[...], 1.0)
    ma = (aggA_ref[...] + aggB_ref[...]) / cnt
    z = (jnp.dot(ma, wl_ref[...], preferred_element_type=jnp.float32)
         + jnp.dot(htb_ref[...], wr_ref[...], preferred_element_type=jnp.float32)
         + b_ref[...])
    z = z - jnp.max(z, axis=1, keepdims=True)
    z = z - jnp.log(jnp.sum(jnp.exp(z), axis=1, keepdims=True))
    out_ref[...] = z


_IB = 1024
_CPB = _IB // _CH


def _make_agg_body(ept, nch, half, garb, z_stripe, out_rows, out_tiles,
                   core_splits_edges):
    nb = ept // _IB
    assert nb % 2 == 0 or nb == 1
    assert nch == nb * _CPB
    cw = (max(z_stripe, out_rows) + 15) // 16 * 16

    def body(tab, src, dst, zr, agg_out, cnt_out,
             sb0, sb1, db0, db1, r0, r1, x0, x1,
             ones_v, cstage_v, agg_sh, cnt_sh, g0, g1, i0, i1):
        srcb = [sb0, sb1]
        dstb = [db0, db1]
        rows = [r0, r1]
        sidx = [x0, x1]
        gsem = [g0, g1]
        isem = [i0, i1]
        c = lax.axis_index("c")
        s = lax.axis_index("s")
        dbase = 0 if core_splits_edges else c * half
        ebase = (c * (ept * _NS) if core_splits_edges else 0) + s * ept

        def zbody(i, carry):
            cstage_v[pl.ds(i * 16, 16)] = jnp.zeros((16,), jnp.float32)
            return carry

        lax.fori_loop(0, cw // 16, zbody, 0)
        pltpu.sync_copy(zr.at[pl.ds(s * z_stripe, z_stripe)],
                        agg_sh.at[pl.ds(s * z_stripe, z_stripe)])
        pltpu.sync_copy(cstage_v.at[pl.ds(0, z_stripe)],
                        cnt_sh.at[pl.ds(s * z_stripe, z_stripe)])
        for k in range(_CH // 16):
            ones_v[pl.ds(k * 16, 16)] = jnp.full((16,), 1.0, jnp.float32)
        pltpu.sync_copy(src.at[pl.ds(ebase, _IB)], srcb[0])
        pltpu.sync_copy(dst.at[pl.ds(ebase, _IB)], dstb[0])
        if nb > 1:
            pltpu.async_copy(src.at[pl.ds(ebase + _IB, _IB)], srcb[1], isem[1])
            pltpu.async_copy(dst.at[pl.ds(ebase + _IB, _IB)], dstb[1], isem[1])
        plsc.subcore_barrier()

        garb_u = jnp.uint32(garb)

        def compute_sidx(dbuf, off, xbuf):
            for k in range(_CH // 16):
                d = dbuf[pl.ds(off + k * 16, 16)]
                loc = plsc.bitcast(d - dbase, jnp.uint32)
                xbuf[pl.ds(k * 16, 16)] = plsc.bitcast(
                    jnp.minimum(loc, garb_u), jnp.int32)

        def gather(sbuf, off, b):
            return pltpu.async_copy(
                tab.at[sbuf.at[pl.ds(off, _CH)]], rows[b], gsem[b])

        compute_sidx(dstb[0], 0, sidx[0])
        gather(srcb[0], 0, 0)
        compute_sidx(dstb[0], _CH, sidx[1])
        gather(srcb[0], _CH, 1)

        def pair_body(t, carry):
            for hm in range(2):
                m = 2 * t + hm
                bufm = hm
                bufn = 1 - hm
                for p in range(_CPB):
                    b = p % 2
                    pltpu.make_async_copy(
                        tab.at[srcb[bufm].at[pl.ds(p * _CH, _CH)]],
                        rows[b], gsem[b]).wait()
                    pltpu.sync_copy(rows[b], agg_sh.at[sidx[b]], add=True)
                    pltpu.sync_copy(ones_v, cnt_sh.at[sidx[b]], add=True)
                    if p == _CPB - 2:
                        @pl.when(m + 2 < nb)
                        def _():
                            off = ebase + (m + 2) * _IB
                            pltpu.async_copy(src.at[pl.ds(off, _IB)],
                                             srcb[bufm], isem[bufm])
                            pltpu.async_copy(dst.at[pl.ds(off, _IB)],
                                             dstb[bufm], isem[bufm])

                        @pl.when(m + 1 < nb)
                        def _():
                            off = ebase + (m + 1) * _IB
                            pltpu.make_async_copy(
                                src.at[pl.ds(off, _IB)], srcb[bufn],
                                isem[bufn]).wait()
                            pltpu.make_async_copy(
                                dst.at[pl.ds(off, _IB)], dstb[bufn],
                                isem[bufn]).wait()
                    j = m * _CPB + p + 2
                    if p < _CPB - 2:
                        jbuf, joff = bufm, (p + 2) * _CH
                    else:
                        jbuf, joff = bufn, (p + 2 - _CPB) * _CH

                    @pl.when(j < nch)
                    def _():
                        compute_sidx(dstb[jbuf], joff, sidx[b])
                        gather(srcb[jbuf], joff, b)
            return carry

        lax.fori_loop(0, max(nb // 2, 1), pair_body, 0)
        plsc.subcore_barrier()

        @pl.when(s < out_tiles)
        def _():
            pltpu.sync_copy(agg_sh.at[pl.ds(s * out_rows, out_rows)],
                            agg_out.at[pl.ds(c * half + s * out_rows, out_rows)])
            pltpu.sync_copy(cnt_sh.at[pl.ds(s * out_rows, out_rows)],
                            cstage_v.at[pl.ds(0, out_rows)])
            pltpu.sync_copy(cstage_v.at[pl.ds(0, out_rows)],
                            cnt_out.at[pl.ds(c * half + s * out_rows, out_rows)])

    return body


def _agg_scratch(rows, cstage):
    return ([pltpu.VMEM((_IB,), jnp.int32)] * 2
            + [pltpu.VMEM((_IB,), jnp.int32)] * 2
            + [pltpu.VMEM((_CH, _IN_CH), jnp.float32)] * 2
            + [pltpu.VMEM((_CH,), jnp.int32)] * 2
            + [pltpu.VMEM((_CH,), jnp.float32),
               pltpu.VMEM((cstage,), jnp.float32),
               pltpu.VMEM_SHARED((rows, _IN_CH), jnp.float32),
               pltpu.VMEM_SHARED((rows,), jnp.float32)]
            + [pltpu.SemaphoreType.DMA] * 4)


@functools.cache
def _sc_kernels():
    mesh = plsc.VectorSubcoreMesh(core_axis_name="c", subcore_axis_name="s",
                                  num_cores=_NC, num_subcores=_NS)
    agg0 = pl.kernel(
        _make_agg_body(_EPT0, _NCH0, _HALF0, _GARB0, z_stripe=640,
                       out_rows=1000, out_tiles=10, core_splits_edges=False),
        out_type=[jax.ShapeDtypeStruct((_ND0, _IN_CH), jnp.float32),
                  jax.ShapeDtypeStruct((_ND0,), jnp.float32)],
        mesh=mesh,
        scratch_types=_agg_scratch(_ROWS0, 1008),
    )
    agg1 = pl.kernel(
        _make_agg_body(_EPT1, _NCH1, _ND1, _ND1 - 1, z_stripe=256,
                       out_rows=256, out_tiles=_NS, core_splits_edges=True),
        out_type=[jax.ShapeDtypeStruct((_NC * _ND1, _HID), jnp.float32),
                  jax.ShapeDtypeStruct((_NC * _ND1,), jnp.float32)],
        mesh=mesh,
        scratch_types=_agg_scratch(_ND1, 256),
    )
    return agg0, agg1


def kernel(x, edge_index0, edge_index1, size0_dst, size1_dst,
           W_l0, W_r0, b0, W_l1, W_r1, b1):
    f32 = jnp.float32
    x = x.astype(f32)
    src0 = edge_index0[0]
    dst0 = edge_index0[1]
    src1 = edge_index1[0]
    dst1 = edge_index1[1]

    sums = pl.pallas_call(
        _colsum_body,
        grid=(_N_SRC // _CBLK,),
        in_specs=[pl.BlockSpec((_CBLK, _IN_CH), lambda i: (i, 0))],
        out_specs=pl.BlockSpec((8, _IN_CH), lambda i: (0, 0)),
        out_shape=jax.ShapeDtypeStruct((8, _IN_CH), f32),
    )(x)

    xb, xtb = pl.pallas_call(
        functools.partial(_binarize_body, nfull=float(_N_SRC), npart=float(_ND0)),
        grid=(_ND0 // _BBLK,),
        in_specs=[pl.BlockSpec((8, _IN_CH), lambda i: (0, 0)),
                  pl.BlockSpec((_BBLK, _IN_CH), lambda i: (i, 0))],
        out_specs=[pl.BlockSpec((_BBLK, _IN_CH), lambda i: (i, 0))] * 2,
        out_shape=[jax.ShapeDtypeStruct((_ND0, _IN_CH), f32)] * 2,
    )(sums, x)

    zr = jnp.zeros((_ROWS0, _IN_CH), f32)
    npad = _E0P - _E0
    src0p = jnp.concatenate([src0, jnp.zeros((npad,), jnp.int32)])
    dst0p = jnp.concatenate([dst0, jnp.full((npad,), _ND0, jnp.int32)])
    sc_agg0, sc_agg1 = _sc_kernels()
    agg0, cnt0 = sc_agg0(xb, src0p, dst0p, zr)

    h, hsums = pl.pallas_call(
        _layer0_body,
        grid=(_ND0 // _HBLK,),
        in_specs=[pl.BlockSpec((_HBLK, _IN_CH), lambda i: (i, 0)),
                  pl.BlockSpec((_HBLK, 1), lambda i: (i, 0)),
                  pl.BlockSpec((_HBLK, _IN_CH), lambda i: (i, 0)),
                  pl.BlockSpec((_IN_CH, _HID), lambda i: (0, 0)),
                  pl.BlockSpec((_IN_CH, _HID), lambda i: (0, 0)),
                  pl.BlockSpec((1, _HID), lambda i: (0, 0))],
        out_specs=[pl.BlockSpec((_HBLK, _HID), lambda i: (i, 0)),
                   pl.BlockSpec((8, _HID), lambda i: (0, 0))],
        out_shape=[jax.ShapeDtypeStruct((_ND0, _HID), f32),
                   jax.ShapeDtypeStruct((8, _HID), f32)],
    )(agg0, cnt0.reshape(_ND0, 1), xtb, W_l0, W_r0, b0.reshape(1, _HID))

    hb, htb = pl.pallas_call(
        functools.partial(_binarize_body, nfull=float(_ND0), npart=float(_ND1)),
        grid=(_ND1 // _FBLK,),
        in_specs=[pl.BlockSpec((8, _HID), lambda i: (0, 0)),
                  pl.BlockSpec((_FBLK, _HID), lambda i: (i, 0))],
        out_specs=[pl.BlockSpec((_FBLK, _HID), lambda i: (i, 0))] * 2,
        out_shape=[jax.ShapeDtypeStruct((_ND1, _HID), f32)] * 2,
    )(hsums, h)

    agg1p, cnt1p = sc_agg1(hb, src1, dst1, zr)

    out = pl.pallas_call(
        _final_body,
        grid=(_ND1 // _FBLK,),
        in_specs=[pl.BlockSpec((_FBLK, _HID), lambda i: (i, 0)),
                  pl.BlockSpec((_FBLK, _HID), lambda i: (i, 0)),
                  pl.BlockSpec((_FBLK, 1), lambda i: (i, 0)),
                  pl.BlockSpec((_FBLK, 1), lambda i: (i, 0)),
                  pl.BlockSpec((_FBLK, _HID), lambda i: (i, 0)),
                  pl.BlockSpec((_HID, _OUT), lambda i: (0, 0)),
                  pl.BlockSpec((_HID, _OUT), lambda i: (0, 0)),
                  pl.BlockSpec((1, _OUT), lambda i: (0, 0))],
        out_specs=pl.BlockSpec((_FBLK, _OUT), lambda i: (i, 0)),
        out_shape=jax.ShapeDtypeStruct((_ND1, _OUT), f32),
    )(agg1p[:_ND1], agg1p[_ND1:], cnt1p[:_ND1].reshape(_ND1, 1),
      cnt1p[_ND1:].reshape(_ND1, 1), htb, W_l1, W_r1, b1.reshape(1, _OUT))
    return out


# speedup vs baseline: 1.5183x; 1.5183x over previous
import functools

import jax
import jax.numpy as jnp
from jax import lax
from jax.experimental import pallas as pl
from jax.experimental.pallas import tpu as pltpu
from jax.experimental.pallas import tpu_sc as plsc

_IN_CH = 128
_HID = 128
_OUT = 64
_N_SRC = 100000
_ND0 = 20000
_ND1 = 4096
_E0 = 320000
_E1 = 65536
_NC = 2
_NS = 16

_CH = 64
_NRING = 4

_HALF0 = _ND0 // 2
_ROWS0 = 10240
_GARB0 = _HALF0
_EPT0 = 20480
_E0P = _EPT0 * _NS
_NCH0 = _EPT0 // _CH

_EPC1 = _E1 // _NC
_EPT1 = _EPC1 // _NS
_NCH1 = _EPT1 // _CH

_CBLK = 2000
_BBLK = 1000
_HBLK = 400
_FBLK = 512


def _colsum_body(x_ref, out_ref):
    i = pl.program_id(0)

    @pl.when(i == 0)
    def _():
        out_ref[...] = jnp.zeros_like(out_ref)

    ssum = jnp.sum(x_ref[...], axis=0, keepdims=True)
    out_ref[0:1, :] += ssum

    @pl.when(i < _ND0 // _CBLK)
    def _():
        out_ref[1:2, :] += ssum


def _binarize_body(sums_ref, x_ref, xb_ref, xtb_ref, *, nfull, npart):
    m_full = sums_ref[0:1, :] / nfull
    m_part = sums_ref[1:2, :] / npart
    blk = x_ref[...]
    xb_ref[...] = jnp.sign(blk - m_full)
    xtb_ref[...] = jnp.sign(blk - m_part)


def _layer0_body(aggA_ref, aggB_ref, cnt_ref, xtb_ref, wlA_ref, wlB_ref,
                 wr_ref, b_ref, h_ref, hsums_ref):
    i = pl.program_id(0)

    @pl.when(i == 0)
    def _():
        hsums_ref[...] = jnp.zeros_like(hsums_ref)

    cnt = jnp.maximum(cnt_ref[...], 1.0)
    hblk = (jnp.dot(aggA_ref[...] / cnt, wlA_ref[...],
                    preferred_element_type=jnp.float32)
            + jnp.dot(aggB_ref[...] / cnt, wlB_ref[...],
                      preferred_element_type=jnp.float32)
            + jnp.dot(xtb_ref[...], wr_ref[...], preferred_element_type=jnp.float32)
            + b_ref[...])
    hblk = jnp.maximum(hblk, 0.0)
    h_ref[...] = hblk
    hsums_ref[0:1, :] += jnp.sum(hblk, axis=0, keepdims=True)
    rows = i * _HBLK + lax.broadcasted_iota(jnp.int32, (_HBLK, 1), 0)
    hsums_ref[1:2, :] += jnp.sum(jnp.where(rows < _ND1, hblk, 0.0),
                                 axis=0, keepdims=True)


def _final_body(aggA_ref, aggB_ref, cntA_ref, cntB_ref, htb_ref,
                wl_ref, wr_ref, b_ref, out_ref):
    cnt = jnp.maximum(cntA_ref[...] + cntB_ref[...], 1.0)
    ma = (aggA_ref[...] + aggB_ref[...]) / cnt
    z = (jnp.dot(ma, wl_ref[...], preferred_element_type=jnp.float32)
         + jnp.dot(htb_ref[...], wr_ref[...], preferred_element_type=jnp.float32)
         + b_ref[...])
    z = z - jnp.max(z, axis=1, keepdims=True)
    z = z - jnp.log(jnp.sum(jnp.exp(z), axis=1, keepdims=True))
    out_ref[...] = z


_IB = 1024
_CPB = _IB // _CH
_CAP0 = _EPT0 + 2 * _CH


def _make_agg_body(ept, nch, half, garb, z_stripe, out_rows, out_tiles,
                   core_splits_edges, col_split=False):
    nb = ept // _IB
    assert nb % 2 == 0 or nb == 1
    assert nch == nb * _CPB
    cw = (max(z_stripe, out_rows) + 15) // 16 * 16

    def body(tab, src, dst, zr, agg_out, cnt_out,
             sb0, sb1, db0, db1, r0, r1, x0, x1, gx0, gx1,
             ones_v, cstage_v, agg_sh, cnt_sh, g0, g1, i0, i1):
        srcb = [sb0, sb1]
        dstb = [db0, db1]
        rows = [r0, r1]
        sidx = [x0, x1]
        gidx = [gx0, gx1]
        gsem = [g0, g1]
        isem = [i0, i1]
        c = lax.axis_index("c")
        s = lax.axis_index("s")
        dbase = 0 if (core_splits_edges or col_split) else c * half
        ebase = (c * (ept * _NS) if core_splits_edges else 0) + s * ept

        def zbody(i, carry):
            cstage_v[pl.ds(i * 16, 16)] = jnp.zeros((16,), jnp.float32)
            return carry

        lax.fori_loop(0, cw // 16, zbody, 0)
        pltpu.sync_copy(zr.at[pl.ds(s * z_stripe, z_stripe)],
                        agg_sh.at[pl.ds(s * z_stripe, z_stripe)])
        pltpu.sync_copy(cstage_v.at[pl.ds(0, z_stripe)],
                        cnt_sh.at[pl.ds(s * z_stripe, z_stripe)])
        for k in range(_CH // 16):
            ones_v[pl.ds(k * 16, 16)] = jnp.full((16,), 1.0, jnp.float32)
        pltpu.sync_copy(src.at[pl.ds(ebase, _IB)], srcb[0])
        pltpu.sync_copy(dst.at[pl.ds(ebase, _IB)], dstb[0])
        if nb > 1:
            pltpu.async_copy(src.at[pl.ds(ebase + _IB, _IB)], srcb[1], isem[1])
            pltpu.async_copy(dst.at[pl.ds(ebase + _IB, _IB)], dstb[1], isem[1])
        plsc.subcore_barrier()

        garb_u = jnp.uint32(garb)

        def compute_sidx(dbuf, off, xbuf):
            for k in range(_CH // 16):
                d = dbuf[pl.ds(off + k * 16, 16)]
                loc = plsc.bitcast(d - dbase, jnp.uint32)
                xbuf[pl.ds(k * 16, 16)] = plsc.bitcast(
                    jnp.minimum(loc, garb_u), jnp.int32)

        def gather(sbuf, off, b):
            if not col_split:
                return pltpu.async_copy(
                    tab.at[sbuf.at[pl.ds(off, _CH)]], rows[b], gsem[b])
            for k in range(_CH // 16):
                sv = sbuf[pl.ds(off + k * 16, 16)]
                gidx[b][pl.ds(k * 16, 16)] = (sv << 1) + c
            return pltpu.async_copy(tab.at[gidx[b]], rows[b], gsem[b])

        def gather_wait(sbuf, off, b):
            if not col_split:
                pltpu.make_async_copy(
                    tab.at[sbuf.at[pl.ds(off, _CH)]], rows[b], gsem[b]).wait()
            else:
                pltpu.make_async_copy(tab.at[gidx[b]], rows[b], gsem[b]).wait()

        def scatter(b):
            pltpu.sync_copy(rows[b], agg_sh.at[sidx[b]], add=True)
            if col_split:
                @pl.when(c == 0)
                def _():
                    pltpu.sync_copy(ones_v, cnt_sh.at[sidx[b]], add=True)
            else:
                pltpu.sync_copy(ones_v, cnt_sh.at[sidx[b]], add=True)

        compute_sidx(dstb[0], 0, sidx[0])
        gather(srcb[0], 0, 0)
        compute_sidx(dstb[0], _CH, sidx[1])
        gather(srcb[0], _CH, 1)

        def pair_body(t, carry):
            for hm in range(2):
                m = 2 * t + hm
                bufm = hm
                bufn = 1 - hm
                for p in range(_CPB):
                    b = p % 2
                    gather_wait(srcb[bufm], p * _CH, b)
                    scatter(b)
                    if p == _CPB - 2:
                        @pl.when(m + 2 < nb)
                        def _():
                            off = ebase + (m + 2) * _IB
                            pltpu.async_copy(src.at[pl.ds(off, _IB)],
                                             srcb[bufm], isem[bufm])
                            pltpu.async_copy(dst.at[pl.ds(off, _IB)],
                                             dstb[bufm], isem[bufm])

                        @pl.when(m + 1 < nb)
                        def _():
                            off = ebase + (m + 1) * _IB
                            pltpu.make_async_copy(
                                src.at[pl.ds(off, _IB)], srcb[bufn],
                                isem[bufn]).wait()
                            pltpu.make_async_copy(
                                dst.at[pl.ds(off, _IB)], dstb[bufn],
                                isem[bufn]).wait()
                    j = m * _CPB + p + 2
                    if p < _CPB - 2:
                        jbuf, joff = bufm, (p + 2) * _CH
                    else:
                        jbuf, joff = bufn, (p + 2 - _CPB) * _CH

                    @pl.when(j < nch)
                    def _():
                        compute_sidx(dstb[jbuf], joff, sidx[b])
                        gather(srcb[jbuf], joff, b)
            return carry

        lax.fori_loop(0, max(nb // 2, 1), pair_body, 0)
        plsc.subcore_barrier()

        @pl.when(s < out_tiles)
        def _():
            pltpu.sync_copy(agg_sh.at[pl.ds(s * out_rows, out_rows)],
                            agg_out.at[pl.ds(c * half + s * out_rows, out_rows)])

        cnt_base = (s * out_rows) if col_split else (c * half + s * out_rows)
        cnt_write = ((c == 0) & (s < out_tiles)) if col_split else (s < out_tiles)

        @pl.when(cnt_write)
        def _():
            pltpu.sync_copy(cnt_sh.at[pl.ds(s * out_rows, out_rows)],
                            cstage_v.at[pl.ds(0, out_rows)])
            pltpu.sync_copy(cstage_v.at[pl.ds(0, out_rows)],
                            cnt_out.at[pl.ds(cnt_base, out_rows)])

    return body


def _agg_scratch(rows, cstage, w):
    return ([pltpu.VMEM((_IB,), jnp.int32)] * 2
            + [pltpu.VMEM((_IB,), jnp.int32)] * 2
            + [pltpu.VMEM((_CH, w), jnp.float32)] * 2
            + [pltpu.VMEM((_CH,), jnp.int32)] * 2
            + [pltpu.VMEM((_CH,), jnp.int32)] * 2
            + [pltpu.VMEM((_CH,), jnp.float32),
               pltpu.VMEM((cstage,), jnp.float32),
               pltpu.VMEM_SHARED((rows, w), jnp.float32),
               pltpu.VMEM_SHARED((rows,), jnp.float32)]
            + [pltpu.SemaphoreType.DMA] * 4)


_ROWS0C = 20480


@functools.cache
def _sc_kernels():
    mesh = plsc.VectorSubcoreMesh(core_axis_name="c", subcore_axis_name="s",
                                  num_cores=_NC, num_subcores=_NS)
    agg0 = pl.kernel(
        _make_agg_body(_EPT0, _NCH0, _ND0, _ROWS0C - 1, z_stripe=1280,
                       out_rows=2000, out_tiles=10, core_splits_edges=False,
                       col_split=True),
        out_type=[jax.ShapeDtypeStruct((_NC * _ND0, _IN_CH // 2), jnp.float32),
                  jax.ShapeDtypeStruct((_ND0,), jnp.float32)],
        mesh=mesh,
        scratch_types=_agg_scratch(_ROWS0C, 2000, _IN_CH // 2),
        compiler_params=pltpu.CompilerParams(use_tc_tiling_on_sc=False),
    )
    agg1 = pl.kernel(
        _make_agg_body(_EPT1, _NCH1, _ND1, _ND1 - 1, z_stripe=256,
                       out_rows=256, out_tiles=_NS, core_splits_edges=True),
        out_type=[jax.ShapeDtypeStruct((_NC * _ND1, _HID), jnp.float32),
                  jax.ShapeDtypeStruct((_NC * _ND1,), jnp.float32)],
        mesh=mesh,
        scratch_types=_agg_scratch(_ND1, 256, _HID),
    )
    return agg0, agg1


def kernel(x, edge_index0, edge_index1, size0_dst, size1_dst,
           W_l0, W_r0, b0, W_l1, W_r1, b1):
    f32 = jnp.float32
    x = x.astype(f32)
    src0 = edge_index0[0]
    dst0 = edge_index0[1]
    src1 = edge_index1[0]
    dst1 = edge_index1[1]

    sums = pl.pallas_call(
        _colsum_body,
        grid=(_N_SRC // _CBLK,),
        in_specs=[pl.BlockSpec((_CBLK, _IN_CH), lambda i: (i, 0))],
        out_specs=pl.BlockSpec((8, _IN_CH), lambda i: (0, 0)),
        out_shape=jax.ShapeDtypeStruct((8, _IN_CH), f32),
    )(x)

    xb, xtb = pl.pallas_call(
        functools.partial(_binarize_body, nfull=float(_N_SRC), npart=float(_ND0)),
        grid=(_ND0 // _BBLK,),
        in_specs=[pl.BlockSpec((8, _IN_CH), lambda i: (0, 0)),
                  pl.BlockSpec((_BBLK, _IN_CH), lambda i: (i, 0))],
        out_specs=[pl.BlockSpec((_BBLK, _IN_CH), lambda i: (i, 0))] * 2,
        out_shape=[jax.ShapeDtypeStruct((_ND0, _IN_CH), f32)] * 2,
    )(sums, x)

    zr0 = jnp.zeros((_ROWS0C, _IN_CH // 2), f32)
    zr1 = jnp.zeros((_NS * 256, _HID), f32)
    npad = _E0P - _E0
    src0p = jnp.concatenate([src0, jnp.zeros((npad,), jnp.int32)])
    dst0p = jnp.concatenate([dst0, jnp.full((npad,), _ND0, jnp.int32)])
    sc_agg0, sc_agg1 = _sc_kernels()
    agg0p, cnt0 = sc_agg0(xb.reshape(_NC * _ND0, _IN_CH // 2), src0p, dst0p,
                          zr0)

    h, hsums = pl.pallas_call(
        _layer0_body,
        grid=(_ND0 // _HBLK,),
        in_specs=[pl.BlockSpec((_HBLK, _IN_CH // 2), lambda i: (i, 0)),
                  pl.BlockSpec((_HBLK, _IN_CH // 2), lambda i: (i, 0)),
                  pl.BlockSpec((_HBLK, 1), lambda i: (i, 0)),
                  pl.BlockSpec((_HBLK, _IN_CH), lambda i: (i, 0)),
                  pl.BlockSpec((_IN_CH // 2, _HID), lambda i: (0, 0)),
                  pl.BlockSpec((_IN_CH // 2, _HID), lambda i: (0, 0)),
                  pl.BlockSpec((_IN_CH, _HID), lambda i: (0, 0)),
                  pl.BlockSpec((1, _HID), lambda i: (0, 0))],
        out_specs=[pl.BlockSpec((_HBLK, _HID), lambda i: (i, 0)),
                   pl.BlockSpec((8, _HID), lambda i: (0, 0))],
        out_shape=[jax.ShapeDtypeStruct((_ND0, _HID), f32),
                   jax.ShapeDtypeStruct((8, _HID), f32)],
    )(agg0p[:_ND0], agg0p[_ND0:], cnt0.reshape(_ND0, 1), xtb,
      W_l0[:_IN_CH // 2], W_l0[_IN_CH // 2:], W_r0, b0.reshape(1, _HID))

    hb, htb = pl.pallas_call(
        functools.partial(_binarize_body, nfull=float(_ND0), npart=float(_ND1)),
        grid=(_ND1 // _FBLK,),
        in_specs=[pl.BlockSpec((8, _HID), lambda i: (0, 0)),
                  pl.BlockSpec((_FBLK, _HID), lambda i: (i, 0))],
        out_specs=[pl.BlockSpec((_FBLK, _HID), lambda i: (i, 0))] * 2,
        out_shape=[jax.ShapeDtypeStruct((_ND1, _HID), f32)] * 2,
    )(hsums, h)

    agg1p, cnt1p = sc_agg1(hb, src1, dst1, zr1)

    out = pl.pallas_call(
        _final_body,
        grid=(_ND1 // _FBLK,),
        in_specs=[pl.BlockSpec((_FBLK, _HID), lambda i: (i, 0)),
                  pl.BlockSpec((_FBLK, _HID), lambda i: (i, 0)),
                  pl.BlockSpec((_FBLK, 1), lambda i: (i, 0)),
                  pl.BlockSpec((_FBLK, 1), lambda i: (i, 0)),
                  pl.BlockSpec((_FBLK, _HID), lambda i: (i, 0)),
                  pl.BlockSpec((_HID, _OUT), lambda i: (0, 0)),
                  pl.BlockSpec((_HID, _OUT), lambda i: (0, 0)),
                  pl.BlockSpec((1, _OUT), lambda i: (0, 0))],
        out_specs=pl.BlockSpec((_FBLK, _OUT), lambda i: (i, 0)),
        out_shape=jax.ShapeDtypeStruct((_ND1, _OUT), f32),
    )(agg1p[:_ND1], agg1p[_ND1:], cnt1p[:_ND1].reshape(_ND1, 1),
      cnt1p[_ND1:].reshape(_ND1, 1), htb, W_l1, W_r1, b1.reshape(1, _OUT))
    return out


# speedup vs baseline: 1.6682x; 1.0987x over previous
import functools

import jax
import jax.numpy as jnp
from jax import lax
from jax.experimental import pallas as pl
from jax.experimental.pallas import tpu as pltpu
from jax.experimental.pallas import tpu_sc as plsc

_IN_CH = 128
_HID = 128
_OUT = 64
_N_SRC = 100000
_ND0 = 20000
_ND1 = 4096
_E0 = 320000
_E1 = 65536
_NC = 2
_NS = 16

_CH = 128
_NRING = 4

_HALF0 = _ND0 // 2
_ROWS0 = 10240
_GARB0 = _HALF0
_EPT0 = 20480
_E0P = _EPT0 * _NS
_NCH0 = _EPT0 // _CH

_EPC1 = _E1 // _NC
_EPT1 = _EPC1 // _NS
_NCH1 = _EPT1 // _CH

_CBLK = 2000
_BBLK = 1000
_HBLK = 400
_FBLK = 512


def _colsum_body(x_ref, out_ref):
    i = pl.program_id(0)

    @pl.when(i == 0)
    def _():
        out_ref[...] = jnp.zeros_like(out_ref)

    ssum = jnp.sum(x_ref[...], axis=0, keepdims=True)
    out_ref[0:1, :] += ssum

    @pl.when(i < _ND0 // _CBLK)
    def _():
        out_ref[1:2, :] += ssum


def _binarize_body(sums_ref, x_ref, xb_ref, xtb_ref, *, nfull, npart):
    m_full = sums_ref[0:1, :] / nfull
    m_part = sums_ref[1:2, :] / npart
    blk = x_ref[...]
    xb_ref[...] = jnp.sign(blk - m_full)
    xtb_ref[...] = jnp.sign(blk - m_part)


def _layer0_body(aggA_ref, aggB_ref, cnt_ref, xtb_ref, wlA_ref, wlB_ref,
                 wr_ref, b_ref, h_ref, hsums_ref):
    i = pl.program_id(0)

    @pl.when(i == 0)
    def _():
        hsums_ref[...] = jnp.zeros_like(hsums_ref)

    cnt = jnp.maximum(cnt_ref[...], 1.0)
    hblk = (jnp.dot(aggA_ref[...] / cnt, wlA_ref[...],
                    preferred_element_type=jnp.float32)
            + jnp.dot(aggB_ref[...] / cnt, wlB_ref[...],
                      preferred_element_type=jnp.float32)
            + jnp.dot(xtb_ref[...], wr_ref[...], preferred_element_type=jnp.float32)
            + b_ref[...])
    hblk = jnp.maximum(hblk, 0.0)
    h_ref[...] = hblk
    hsums_ref[0:1, :] += jnp.sum(hblk, axis=0, keepdims=True)
    rows = i * _HBLK + lax.broadcasted_iota(jnp.int32, (_HBLK, 1), 0)
    hsums_ref[1:2, :] += jnp.sum(jnp.where(rows < _ND1, hblk, 0.0),
                                 axis=0, keepdims=True)


def _final_body(aggA_ref, aggB_ref, cntA_ref, cntB_ref, htb_ref,
                wl_ref, wr_ref, b_ref, out_ref):
    cnt = jnp.maximum(cntA_ref[...] + cntB_ref[...], 1.0)
    ma = (aggA_ref[...] + aggB_ref[...]) / cnt
    z = (jnp.dot(ma, wl_ref[...], preferred_element_type=jnp.float32)
         + jnp.dot(htb_ref[...], wr_ref[...], preferred_element_type=jnp.float32)
         + b_ref[...])
    z = z - jnp.max(z, axis=1, keepdims=True)
    z = z - jnp.log(jnp.sum(jnp.exp(z), axis=1, keepdims=True))
    out_ref[...] = z


_IB = 1024
_CPB = _IB // _CH
_CAP0 = _EPT0 + 2 * _CH


def _make_agg_body(ept, nch, half, garb, z_stripe, out_rows, out_tiles,
                   core_splits_edges, col_split=False):
    nb = ept // _IB
    assert nb % 2 == 0 or nb == 1
    assert nch == nb * _CPB
    assert _CPB >= _NRING
    cw = (max(z_stripe, out_rows) + 15) // 16 * 16

    def body(tab, src, dst, zr, agg_out, cnt_out, *sc):
        srcb = [sc[0], sc[1]]
        dstb = [sc[2], sc[3]]
        rows = list(sc[4:4 + _NRING])
        sidx = list(sc[4 + _NRING:4 + 2 * _NRING])
        gidx = list(sc[4 + 2 * _NRING:4 + 3 * _NRING])
        ones_v, cstage_v, agg_sh, cnt_sh = sc[4 + 3 * _NRING:8 + 3 * _NRING]
        gsem = list(sc[8 + 3 * _NRING:8 + 4 * _NRING])
        isem = list(sc[8 + 4 * _NRING:10 + 4 * _NRING])
        c = lax.axis_index("c")
        s = lax.axis_index("s")
        dbase = 0 if (core_splits_edges or col_split) else c * half
        ebase = (c * (ept * _NS) if core_splits_edges else 0) + s * ept

        def zbody(i, carry):
            cstage_v[pl.ds(i * 16, 16)] = jnp.zeros((16,), jnp.float32)
            return carry

        lax.fori_loop(0, cw // 16, zbody, 0)
        pltpu.sync_copy(zr.at[pl.ds(s * z_stripe, z_stripe)],
                        agg_sh.at[pl.ds(s * z_stripe, z_stripe)])
        pltpu.sync_copy(cstage_v.at[pl.ds(0, z_stripe)],
                        cnt_sh.at[pl.ds(s * z_stripe, z_stripe)])
        for k in range(_CH // 16):
            ones_v[pl.ds(k * 16, 16)] = jnp.full((16,), 1.0, jnp.float32)
        pltpu.sync_copy(src.at[pl.ds(ebase, _IB)], srcb[0])
        pltpu.sync_copy(dst.at[pl.ds(ebase, _IB)], dstb[0])
        if nb > 1:
            pltpu.async_copy(src.at[pl.ds(ebase + _IB, _IB)], srcb[1], isem[1])
            pltpu.async_copy(dst.at[pl.ds(ebase + _IB, _IB)], dstb[1], isem[1])
        plsc.subcore_barrier()

        garb_u = jnp.uint32(garb)

        def compute_sidx(dbuf, off, xbuf):
            for k in range(_CH // 16):
                d = dbuf[pl.ds(off + k * 16, 16)]
                loc = plsc.bitcast(d - dbase, jnp.uint32)
                xbuf[pl.ds(k * 16, 16)] = plsc.bitcast(
                    jnp.minimum(loc, garb_u), jnp.int32)

        def gather(sbuf, off, b):
            if not col_split:
                return pltpu.async_copy(
                    tab.at[sbuf.at[pl.ds(off, _CH)]], rows[b], gsem[b])
            for k in range(_CH // 16):
                sv = sbuf[pl.ds(off + k * 16, 16)]
                gidx[b][pl.ds(k * 16, 16)] = (sv << 1) + c
            return pltpu.async_copy(tab.at[gidx[b]], rows[b], gsem[b])

        def gather_wait(sbuf, off, b):
            if not col_split:
                pltpu.make_async_copy(
                    tab.at[sbuf.at[pl.ds(off, _CH)]], rows[b], gsem[b]).wait()
            else:
                pltpu.make_async_copy(tab.at[gidx[b]], rows[b], gsem[b]).wait()

        def scatter(b):
            pltpu.sync_copy(rows[b], agg_sh.at[sidx[b]], add=True)
            if col_split:
                @pl.when(c == 0)
                def _():
                    pltpu.sync_copy(ones_v, cnt_sh.at[sidx[b]], add=True)
            else:
                pltpu.sync_copy(ones_v, cnt_sh.at[sidx[b]], add=True)

        for b in range(_NRING):
            compute_sidx(dstb[0], b * _CH, sidx[b])
            gather(srcb[0], b * _CH, b)

        def pair_body(t, carry):
            for hm in range(2):
                m = 2 * t + hm
                bufm = hm
                bufn = 1 - hm
                for p in range(_CPB):
                    b = p % _NRING
                    gather_wait(srcb[bufm], p * _CH, b)
                    scatter(b)
                    if p == _CPB - _NRING:
                        @pl.when(m + 2 < nb)
                        def _():
                            off = ebase + (m + 2) * _IB
                            pltpu.async_copy(src.at[pl.ds(off, _IB)],
                                             srcb[bufm], isem[bufm])
                            pltpu.async_copy(dst.at[pl.ds(off, _IB)],
                                             dstb[bufm], isem[bufm])

                        @pl.when(m + 1 < nb)
                        def _():
                            off = ebase + (m + 1) * _IB
                            pltpu.make_async_copy(
                                src.at[pl.ds(off, _IB)], srcb[bufn],
                                isem[bufn]).wait()
                            pltpu.make_async_copy(
                                dst.at[pl.ds(off, _IB)], dstb[bufn],
                                isem[bufn]).wait()
                    j = m * _CPB + p + _NRING
                    if p < _CPB - _NRING:
                        jbuf, joff = bufm, (p + _NRING) * _CH
                    else:
                        jbuf, joff = bufn, (p + _NRING - _CPB) * _CH

                    @pl.when(j < nch)
                    def _():
                        compute_sidx(dstb[jbuf], joff, sidx[b])
                        gather(srcb[jbuf], joff, b)
            return carry

        lax.fori_loop(0, max(nb // 2, 1), pair_body, 0)
        plsc.subcore_barrier()

        @pl.when(s < out_tiles)
        def _():
            pltpu.sync_copy(agg_sh.at[pl.ds(s * out_rows, out_rows)],
                            agg_out.at[pl.ds(c * half + s * out_rows, out_rows)])

        cnt_base = (s * out_rows) if col_split else (c * half + s * out_rows)
        cnt_write = ((c == 0) & (s < out_tiles)) if col_split else (s < out_tiles)

        @pl.when(cnt_write)
        def _():
            pltpu.sync_copy(cnt_sh.at[pl.ds(s * out_rows, out_rows)],
                            cstage_v.at[pl.ds(0, out_rows)])
            pltpu.sync_copy(cstage_v.at[pl.ds(0, out_rows)],
                            cnt_out.at[pl.ds(cnt_base, out_rows)])

    return body


def _agg_scratch(rows, cstage, w):
    return ([pltpu.VMEM((_IB,), jnp.int32)] * 2
            + [pltpu.VMEM((_IB,), jnp.int32)] * 2
            + [pltpu.VMEM((_CH, w), jnp.float32)] * _NRING
            + [pltpu.VMEM((_CH,), jnp.int32)] * _NRING
            + [pltpu.VMEM((_CH,), jnp.int32)] * _NRING
            + [pltpu.VMEM((_CH,), jnp.float32),
               pltpu.VMEM((cstage,), jnp.float32),
               pltpu.VMEM_SHARED((rows, w), jnp.float32),
               pltpu.VMEM_SHARED((rows,), jnp.float32)]
            + [pltpu.SemaphoreType.DMA] * (_NRING + 2))


_ROWS0C = 20480


@functools.cache
def _sc_kernels():
    mesh = plsc.VectorSubcoreMesh(core_axis_name="c", subcore_axis_name="s",
                                  num_cores=_NC, num_subcores=_NS)
    agg0 = pl.kernel(
        _make_agg_body(_EPT0, _NCH0, _ND0, _ROWS0C - 1, z_stripe=1280,
                       out_rows=2000, out_tiles=10, core_splits_edges=False,
                       col_split=True),
        out_type=[jax.ShapeDtypeStruct((_NC * _ND0, _IN_CH // 2), jnp.float32),
                  jax.ShapeDtypeStruct((_ND0,), jnp.float32)],
        mesh=mesh,
        scratch_types=_agg_scratch(_ROWS0C, 2000, _IN_CH // 2),
        compiler_params=pltpu.CompilerParams(use_tc_tiling_on_sc=False),
    )
    agg1 = pl.kernel(
        _make_agg_body(_EPT1, _NCH1, _ND1, _ND1 - 1, z_stripe=256,
                       out_rows=256, out_tiles=_NS, core_splits_edges=True),
        out_type=[jax.ShapeDtypeStruct((_NC * _ND1, _HID), jnp.float32),
                  jax.ShapeDtypeStruct((_NC * _ND1,), jnp.float32)],
        mesh=mesh,
        scratch_types=_agg_scratch(_ND1, 256, _HID),
    )
    return agg0, agg1


def kernel(x, edge_index0, edge_index1, size0_dst, size1_dst,
           W_l0, W_r0, b0, W_l1, W_r1, b1):
    f32 = jnp.float32
    x = x.astype(f32)
    src0 = edge_index0[0]
    dst0 = edge_index0[1]
    src1 = edge_index1[0]
    dst1 = edge_index1[1]

    sums = pl.pallas_call(
        _colsum_body,
        grid=(_N_SRC // _CBLK,),
        in_specs=[pl.BlockSpec((_CBLK, _IN_CH), lambda i: (i, 0))],
        out_specs=pl.BlockSpec((8, _IN_CH), lambda i: (0, 0)),
        out_shape=jax.ShapeDtypeStruct((8, _IN_CH), f32),
    )(x)

    xb, xtb = pl.pallas_call(
        functools.partial(_binarize_body, nfull=float(_N_SRC), npart=float(_ND0)),
        grid=(_ND0 // _BBLK,),
        in_specs=[pl.BlockSpec((8, _IN_CH), lambda i: (0, 0)),
                  pl.BlockSpec((_BBLK, _IN_CH), lambda i: (i, 0))],
        out_specs=[pl.BlockSpec((_BBLK, _IN_CH), lambda i: (i, 0))] * 2,
        out_shape=[jax.ShapeDtypeStruct((_ND0, _IN_CH), f32)] * 2,
    )(sums, x)

    zr0 = jnp.zeros((_ROWS0C, _IN_CH // 2), f32)
    zr1 = jnp.zeros((_NS * 256, _HID), f32)
    npad = _E0P - _E0
    src0p = jnp.concatenate([src0, jnp.zeros((npad,), jnp.int32)])
    dst0p = jnp.concatenate([dst0, jnp.full((npad,), _ND0, jnp.int32)])
    sc_agg0, sc_agg1 = _sc_kernels()
    agg0p, cnt0 = sc_agg0(xb.reshape(_NC * _ND0, _IN_CH // 2), src0p, dst0p,
                          zr0)

    h, hsums = pl.pallas_call(
        _layer0_body,
        grid=(_ND0 // _HBLK,),
        in_specs=[pl.BlockSpec((_HBLK, _IN_CH // 2), lambda i: (i, 0)),
                  pl.BlockSpec((_HBLK, _IN_CH // 2), lambda i: (i, 0)),
                  pl.BlockSpec((_HBLK, 1), lambda i: (i, 0)),
                  pl.BlockSpec((_HBLK, _IN_CH), lambda i: (i, 0)),
                  pl.BlockSpec((_IN_CH // 2, _HID), lambda i: (0, 0)),
                  pl.BlockSpec((_IN_CH // 2, _HID), lambda i: (0, 0)),
                  pl.BlockSpec((_IN_CH, _HID), lambda i: (0, 0)),
                  pl.BlockSpec((1, _HID), lambda i: (0, 0))],
        out_specs=[pl.BlockSpec((_HBLK, _HID), lambda i: (i, 0)),
                   pl.BlockSpec((8, _HID), lambda i: (0, 0))],
        out_shape=[jax.ShapeDtypeStruct((_ND0, _HID), f32),
                   jax.ShapeDtypeStruct((8, _HID), f32)],
    )(agg0p[:_ND0], agg0p[_ND0:], cnt0.reshape(_ND0, 1), xtb,
      W_l0[:_IN_CH // 2], W_l0[_IN_CH // 2:], W_r0, b0.reshape(1, _HID))

    hb, htb = pl.pallas_call(
        functools.partial(_binarize_body, nfull=float(_ND0), npart=float(_ND1)),
        grid=(_ND1 // _FBLK,),
        in_specs=[pl.BlockSpec((8, _HID), lambda i: (0, 0)),
                  pl.BlockSpec((_FBLK, _HID), lambda i: (i, 0))],
        out_specs=[pl.BlockSpec((_FBLK, _HID), lambda i: (i, 0))] * 2,
        out_shape=[jax.ShapeDtypeStruct((_ND1, _HID), f32)] * 2,
    )(hsums, h)

    agg1p, cnt1p = sc_agg1(hb, src1, dst1, zr1)

    out = pl.pallas_call(
        _final_body,
        grid=(_ND1 // _FBLK,),
        in_specs=[pl.BlockSpec((_FBLK, _HID), lambda i: (i, 0)),
                  pl.BlockSpec((_FBLK, _HID), lambda i: (i, 0)),
                  pl.BlockSpec((_FBLK, 1), lambda i: (i, 0)),
                  pl.BlockSpec((_FBLK, 1), lambda i: (i, 0)),
                  pl.BlockSpec((_FBLK, _HID), lambda i: (i, 0)),
                  pl.BlockSpec((_HID, _OUT), lambda i: (0, 0)),
                  pl.BlockSpec((_HID, _OUT), lambda i: (0, 0)),
                  pl.BlockSpec((1, _OUT), lambda i: (0, 0))],
        out_specs=pl.BlockSpec((_FBLK, _OUT), lambda i: (i, 0)),
        out_shape=jax.ShapeDtypeStruct((_ND1, _OUT), f32),
    )(agg1p[:_ND1], agg1p[_ND1:], cnt1p[:_ND1].reshape(_ND1, 1),
      cnt1p[_ND1:].reshape(_ND1, 1), htb, W_l1, W_r1, b1.reshape(1, _OUT))
    return out


# speedup vs baseline: 2.2530x; 1.3505x over previous
import functools

import jax
import jax.numpy as jnp
from jax import lax
from jax.experimental import pallas as pl
from jax.experimental.pallas import tpu as pltpu
from jax.experimental.pallas import tpu_sc as plsc

_IN_CH = 128
_HID = 128
_OUT = 64
_N_SRC = 100000
_ND0 = 20000
_ND1 = 4096
_E0 = 320000
_E1 = 65536
_NC = 2
_NS = 16

_CH = 128
_NRING = 4

_HALF0 = _ND0 // 2
_ROWS0 = 10240
_GARB0 = _HALF0
_EPT0 = 20480
_E0P = _EPT0 * _NS
_NCH0 = _EPT0 // _CH

_EPC1 = _E1 // _NC
_EPT1 = _EPC1 // _NS
_NCH1 = _EPT1 // _CH

_CBLK = 2000
_BBLK = 2000
_HBLK = 400
_FBLK = 512


def _colsum_body(x_ref, out_ref):
    i = pl.program_id(0)

    @pl.when(i == 0)
    def _():
        out_ref[...] = jnp.zeros_like(out_ref)

    ssum = jnp.sum(x_ref[...], axis=0, keepdims=True)
    out_ref[0:1, :] += ssum

    @pl.when(i < _ND0 // _CBLK)
    def _():
        out_ref[1:2, :] += ssum


def _binarize_body(sums_ref, x_ref, xb_ref, xtb_ref, *, nfull, npart):
    m_full = sums_ref[0:1, :] / nfull
    m_part = sums_ref[1:2, :] / npart
    blk = x_ref[...]
    xb_ref[...] = jnp.sign(blk - m_full).astype(xb_ref.dtype)
    xtb_ref[...] = jnp.sign(blk - m_part)


def _layer0_body(aggA_ref, aggB_ref, cnt_ref, xtb_ref, wlA_ref, wlB_ref,
                 wr_ref, b_ref, h_ref, hsums_ref):
    i = pl.program_id(0)

    @pl.when(i == 0)
    def _():
        hsums_ref[...] = jnp.zeros_like(hsums_ref)

    cnt = jnp.maximum(cnt_ref[...], 1.0)
    aggA = aggA_ref[...].astype(jnp.float32)
    aggB = aggB_ref[...].astype(jnp.float32)
    hblk = (jnp.dot(aggA / cnt, wlA_ref[...],
                    preferred_element_type=jnp.float32)
            + jnp.dot(aggB / cnt, wlB_ref[...],
                      preferred_element_type=jnp.float32)
            + jnp.dot(xtb_ref[...], wr_ref[...], preferred_element_type=jnp.float32)
            + b_ref[...])
    hblk = jnp.maximum(hblk, 0.0)
    h_ref[...] = hblk
    hsums_ref[0:1, :] += jnp.sum(hblk, axis=0, keepdims=True)
    rows = i * _HBLK + lax.broadcasted_iota(jnp.int32, (_HBLK, 1), 0)
    hsums_ref[1:2, :] += jnp.sum(jnp.where(rows < _ND1, hblk, 0.0),
                                 axis=0, keepdims=True)


def _final_body(aggA_ref, aggB_ref, cntA_ref, cntB_ref, htb_ref,
                wl_ref, wr_ref, b_ref, out_ref):
    cnt = jnp.maximum(cntA_ref[...] + cntB_ref[...], 1.0)
    ma = (aggA_ref[...] + aggB_ref[...]) / cnt
    z = (jnp.dot(ma, wl_ref[...], preferred_element_type=jnp.float32)
         + jnp.dot(htb_ref[...], wr_ref[...], preferred_element_type=jnp.float32)
         + b_ref[...])
    z = z - jnp.max(z, axis=1, keepdims=True)
    z = z - jnp.log(jnp.sum(jnp.exp(z), axis=1, keepdims=True))
    out_ref[...] = z


_IB = 1024
_CPB = _IB // _CH
_CAP0 = _EPT0 + 2 * _CH


def _make_agg_body(ept, nch, half, garb, z_stripe, out_rows, out_tiles,
                   core_splits_edges, col_split=False):
    nb = ept // _IB
    assert nb % 2 == 0 or nb == 1
    assert nch == nb * _CPB
    assert _CPB >= _NRING
    cw = (max(z_stripe, out_rows) + 15) // 16 * 16

    def body(tab, src, dst, zr, agg_out, cnt_out, *sc):
        srcb = [sc[0], sc[1]]
        dstb = [sc[2], sc[3]]
        rows = list(sc[4:4 + _NRING])
        sidx = list(sc[4 + _NRING:4 + 2 * _NRING])
        gidx = list(sc[4 + 2 * _NRING:4 + 3 * _NRING])
        ones_v, cstage_v, agg_sh, cnt_sh = sc[4 + 3 * _NRING:8 + 3 * _NRING]
        gsem = list(sc[8 + 3 * _NRING:8 + 4 * _NRING])
        isem = list(sc[8 + 4 * _NRING:10 + 4 * _NRING])
        c = lax.axis_index("c")
        s = lax.axis_index("s")
        dbase = 0 if (core_splits_edges or col_split) else c * half
        ebase = (c * (ept * _NS) if core_splits_edges else 0) + s * ept

        def zbody(i, carry):
            cstage_v[pl.ds(i * 16, 16)] = jnp.zeros((16,), jnp.float32)
            return carry

        lax.fori_loop(0, cw // 16, zbody, 0)
        pltpu.sync_copy(zr.at[pl.ds(s * z_stripe, z_stripe)],
                        agg_sh.at[pl.ds(s * z_stripe, z_stripe)])
        pltpu.sync_copy(cstage_v.at[pl.ds(0, z_stripe)],
                        cnt_sh.at[pl.ds(s * z_stripe, z_stripe)])
        for k in range(_CH // 16):
            ones_v[pl.ds(k * 16, 16)] = jnp.full((16,), 1.0, jnp.float32)
        pltpu.sync_copy(src.at[pl.ds(ebase, _IB)], srcb[0])
        pltpu.sync_copy(dst.at[pl.ds(ebase, _IB)], dstb[0])
        if nb > 1:
            pltpu.async_copy(src.at[pl.ds(ebase + _IB, _IB)], srcb[1], isem[1])
            pltpu.async_copy(dst.at[pl.ds(ebase + _IB, _IB)], dstb[1], isem[1])
        plsc.subcore_barrier()

        garb_u = jnp.uint32(garb)

        def compute_sidx(dbuf, off, xbuf):
            for k in range(_CH // 16):
                d = dbuf[pl.ds(off + k * 16, 16)]
                loc = plsc.bitcast(d - dbase, jnp.uint32)
                xbuf[pl.ds(k * 16, 16)] = plsc.bitcast(
                    jnp.minimum(loc, garb_u), jnp.int32)

        def gather(sbuf, off, b):
            if not col_split:
                return pltpu.async_copy(
                    tab.at[sbuf.at[pl.ds(off, _CH)]], rows[b], gsem[b])
            for k in range(_CH // 16):
                sv = sbuf[pl.ds(off + k * 16, 16)]
                gidx[b][pl.ds(k * 16, 16)] = (sv << 1) + c
            return pltpu.async_copy(tab.at[gidx[b]], rows[b], gsem[b])

        def gather_wait(sbuf, off, b):
            if not col_split:
                pltpu.make_async_copy(
                    tab.at[sbuf.at[pl.ds(off, _CH)]], rows[b], gsem[b]).wait()
            else:
                pltpu.make_async_copy(tab.at[gidx[b]], rows[b], gsem[b]).wait()

        def scatter(b):
            pltpu.sync_copy(rows[b], agg_sh.at[sidx[b]], add=True)
            if col_split:
                @pl.when(c == 0)
                def _():
                    pltpu.sync_copy(ones_v, cnt_sh.at[sidx[b]], add=True)
            else:
                pltpu.sync_copy(ones_v, cnt_sh.at[sidx[b]], add=True)

        for b in range(_NRING):
            compute_sidx(dstb[0], b * _CH, sidx[b])
            gather(srcb[0], b * _CH, b)

        def pair_body(t, carry):
            for hm in range(2):
                m = 2 * t + hm
                bufm = hm
                bufn = 1 - hm
                for p in range(_CPB):
                    b = p % _NRING
                    gather_wait(srcb[bufm], p * _CH, b)
                    scatter(b)
                    if p == _CPB - _NRING:
                        @pl.when(m + 2 < nb)
                        def _():
                            off = ebase + (m + 2) * _IB
                            pltpu.async_copy(src.at[pl.ds(off, _IB)],
                                             srcb[bufm], isem[bufm])
                            pltpu.async_copy(dst.at[pl.ds(off, _IB)],
                                             dstb[bufm], isem[bufm])

                        @pl.when(m + 1 < nb)
                        def _():
                            off = ebase + (m + 1) * _IB
                            pltpu.make_async_copy(
                                src.at[pl.ds(off, _IB)], srcb[bufn],
                                isem[bufn]).wait()
                            pltpu.make_async_copy(
                                dst.at[pl.ds(off, _IB)], dstb[bufn],
                                isem[bufn]).wait()
                    j = m * _CPB + p + _NRING
                    if p < _CPB - _NRING:
                        jbuf, joff = bufm, (p + _NRING) * _CH
                    else:
                        jbuf, joff = bufn, (p + _NRING - _CPB) * _CH

                    @pl.when(j < nch)
                    def _():
                        compute_sidx(dstb[jbuf], joff, sidx[b])
                        gather(srcb[jbuf], joff, b)
            return carry

        lax.fori_loop(0, max(nb // 2, 1), pair_body, 0)
        plsc.subcore_barrier()

        @pl.when(s < out_tiles)
        def _():
            pltpu.sync_copy(agg_sh.at[pl.ds(s * out_rows, out_rows)],
                            agg_out.at[pl.ds(c * half + s * out_rows, out_rows)])

        cnt_base = (s * out_rows) if col_split else (c * half + s * out_rows)
        cnt_write = ((c == 0) & (s < out_tiles)) if col_split else (s < out_tiles)

        @pl.when(cnt_write)
        def _():
            pltpu.sync_copy(cnt_sh.at[pl.ds(s * out_rows, out_rows)],
                            cstage_v.at[pl.ds(0, out_rows)])
            pltpu.sync_copy(cstage_v.at[pl.ds(0, out_rows)],
                            cnt_out.at[pl.ds(cnt_base, out_rows)])

    return body


def _agg_scratch(rows, cstage, w, dtype=jnp.float32):
    return ([pltpu.VMEM((_IB,), jnp.int32)] * 2
            + [pltpu.VMEM((_IB,), jnp.int32)] * 2
            + [pltpu.VMEM((_CH, w), dtype)] * _NRING
            + [pltpu.VMEM((_CH,), jnp.int32)] * _NRING
            + [pltpu.VMEM((_CH,), jnp.int32)] * _NRING
            + [pltpu.VMEM((_CH,), jnp.float32),
               pltpu.VMEM((cstage,), jnp.float32),
               pltpu.VMEM_SHARED((rows, w), dtype),
               pltpu.VMEM_SHARED((rows,), jnp.float32)]
            + [pltpu.SemaphoreType.DMA] * (_NRING + 2))


_ROWS0C = 20480


@functools.cache
def _sc_kernels():
    mesh = plsc.VectorSubcoreMesh(core_axis_name="c", subcore_axis_name="s",
                                  num_cores=_NC, num_subcores=_NS)
    agg0 = pl.kernel(
        _make_agg_body(_EPT0, _NCH0, _ND0, _ROWS0C - 1, z_stripe=1280,
                       out_rows=2000, out_tiles=10, core_splits_edges=False,
                       col_split=True),
        out_type=[jax.ShapeDtypeStruct((_NC * _ND0, _IN_CH // 2), jnp.bfloat16),
                  jax.ShapeDtypeStruct((_ND0,), jnp.float32)],
        mesh=mesh,
        scratch_types=_agg_scratch(_ROWS0C, 2000, _IN_CH // 2, jnp.bfloat16),
        compiler_params=pltpu.CompilerParams(use_tc_tiling_on_sc=False),
    )
    agg1 = pl.kernel(
        _make_agg_body(_EPT1, _NCH1, _ND1, _ND1 - 1, z_stripe=256,
                       out_rows=256, out_tiles=_NS, core_splits_edges=True),
        out_type=[jax.ShapeDtypeStruct((_NC * _ND1, _HID), jnp.float32),
                  jax.ShapeDtypeStruct((_NC * _ND1,), jnp.float32)],
        mesh=mesh,
        scratch_types=_agg_scratch(_ND1, 256, _HID),
    )
    return agg0, agg1


def kernel(x, edge_index0, edge_index1, size0_dst, size1_dst,
           W_l0, W_r0, b0, W_l1, W_r1, b1):
    f32 = jnp.float32
    x = x.astype(f32)
    src0 = edge_index0[0]
    dst0 = edge_index0[1]
    src1 = edge_index1[0]
    dst1 = edge_index1[1]

    sums = pl.pallas_call(
        _colsum_body,
        grid=(_N_SRC // _CBLK,),
        in_specs=[pl.BlockSpec((_CBLK, _IN_CH), lambda i: (i, 0))],
        out_specs=pl.BlockSpec((8, _IN_CH), lambda i: (0, 0)),
        out_shape=jax.ShapeDtypeStruct((8, _IN_CH), f32),
    )(x)

    xb, xtb = pl.pallas_call(
        functools.partial(_binarize_body, nfull=float(_N_SRC), npart=float(_ND0)),
        grid=(_ND0 // _BBLK,),
        in_specs=[pl.BlockSpec((8, _IN_CH), lambda i: (0, 0)),
                  pl.BlockSpec((_BBLK, _IN_CH), lambda i: (i, 0))],
        out_specs=[pl.BlockSpec((_BBLK, _IN_CH), lambda i: (i, 0))] * 2,
        out_shape=[jax.ShapeDtypeStruct((_ND0, _IN_CH), jnp.bfloat16),
                   jax.ShapeDtypeStruct((_ND0, _IN_CH), f32)],
    )(sums, x)

    zr0 = jnp.zeros((_ROWS0C, _IN_CH // 2), jnp.bfloat16)
    zr1 = jnp.zeros((_NS * 256, _HID), f32)
    npad = _E0P - _E0
    src0p = jnp.concatenate([src0, jnp.zeros((npad,), jnp.int32)])
    dst0p = jnp.concatenate([dst0, jnp.full((npad,), _ND0, jnp.int32)])
    sc_agg0, sc_agg1 = _sc_kernels()
    agg0p, cnt0 = sc_agg0(xb.reshape(_NC * _ND0, _IN_CH // 2), src0p, dst0p,
                          zr0)

    h, hsums = pl.pallas_call(
        _layer0_body,
        grid=(_ND0 // _HBLK,),
        in_specs=[pl.BlockSpec((_HBLK, _IN_CH // 2), lambda i: (i, 0)),
                  pl.BlockSpec((_HBLK, _IN_CH // 2), lambda i: (i, 0)),
                  pl.BlockSpec((_HBLK, 1), lambda i: (i, 0)),
                  pl.BlockSpec((_HBLK, _IN_CH), lambda i: (i, 0)),
                  pl.BlockSpec((_IN_CH // 2, _HID), lambda i: (0, 0)),
                  pl.BlockSpec((_IN_CH // 2, _HID), lambda i: (0, 0)),
                  pl.BlockSpec((_IN_CH, _HID), lambda i: (0, 0)),
                  pl.BlockSpec((1, _HID), lambda i: (0, 0))],
        out_specs=[pl.BlockSpec((_HBLK, _HID), lambda i: (i, 0)),
                   pl.BlockSpec((8, _HID), lambda i: (0, 0))],
        out_shape=[jax.ShapeDtypeStruct((_ND0, _HID), f32),
                   jax.ShapeDtypeStruct((8, _HID), f32)],
    )(agg0p[:_ND0], agg0p[_ND0:], cnt0.reshape(_ND0, 1), xtb,
      W_l0[:_IN_CH // 2], W_l0[_IN_CH // 2:], W_r0, b0.reshape(1, _HID))

    hb, htb = pl.pallas_call(
        functools.partial(_binarize_body, nfull=float(_ND0), npart=float(_ND1)),
        grid=(_ND1 // _FBLK,),
        in_specs=[pl.BlockSpec((8, _HID), lambda i: (0, 0)),
                  pl.BlockSpec((_FBLK, _HID), lambda i: (i, 0))],
        out_specs=[pl.BlockSpec((_FBLK, _HID), lambda i: (i, 0))] * 2,
        out_shape=[jax.ShapeDtypeStruct((_ND1, _HID), f32)] * 2,
    )(hsums, h)

    agg1p, cnt1p = sc_agg1(hb, src1, dst1, zr1)

    out = pl.pallas_call(
        _final_body,
        grid=(_ND1 // _FBLK,),
        in_specs=[pl.BlockSpec((_FBLK, _HID), lambda i: (i, 0)),
                  pl.BlockSpec((_FBLK, _HID), lambda i: (i, 0)),
                  pl.BlockSpec((_FBLK, 1), lambda i: (i, 0)),
                  pl.BlockSpec((_FBLK, 1), lambda i: (i, 0)),
                  pl.BlockSpec((_FBLK, _HID), lambda i: (i, 0)),
                  pl.BlockSpec((_HID, _OUT), lambda i: (0, 0)),
                  pl.BlockSpec((_HID, _OUT), lambda i: (0, 0)),
                  pl.BlockSpec((1, _OUT), lambda i: (0, 0))],
        out_specs=pl.BlockSpec((_FBLK, _OUT), lambda i: (i, 0)),
        out_shape=jax.ShapeDtypeStruct((_ND1, _OUT), f32),
    )(agg1p[:_ND1], agg1p[_ND1:], cnt1p[:_ND1].reshape(_ND1, 1),
      cnt1p[_ND1:].reshape(_ND1, 1), htb, W_l1, W_r1, b1.reshape(1, _OUT))
    return out


# speedup vs baseline: 2.2787x; 1.0114x over previous
import functools

import jax
import jax.numpy as jnp
from jax import lax
from jax.experimental import pallas as pl
from jax.experimental.pallas import tpu as pltpu
from jax.experimental.pallas import tpu_sc as plsc

_IN_CH = 128
_HID = 128
_OUT = 64
_N_SRC = 100000
_ND0 = 20000
_ND1 = 4096
_E0 = 320000
_E1 = 65536
_NC = 2
_NS = 16

_CH = 128
_NRING = 4

_HALF0 = _ND0 // 2
_ROWS0 = 10240
_GARB0 = _HALF0
_EPT0 = 20480
_E0P = _EPT0 * _NS
_NCH0 = _EPT0 // _CH

_EPC1 = _E1 // _NC
_EPT1 = _EPC1 // _NS
_NCH1 = _EPT1 // _CH

_CBLK = 2000
_BBLK = 2000
_HBLK = 400
_FBLK = 512


def _colsum_body(x_ref, out_ref):
    i = pl.program_id(0)

    @pl.when(i == 0)
    def _():
        out_ref[...] = jnp.zeros_like(out_ref)

    ssum = jnp.sum(x_ref[...], axis=0, keepdims=True)
    out_ref[0:1, :] += ssum

    @pl.when(i < _ND0 // _CBLK)
    def _():
        out_ref[1:2, :] += ssum


def _binarize_body(sums_ref, x_ref, xb_ref, xtb_ref, *, nfull, npart):
    m_full = sums_ref[0:1, :] / nfull
    m_part = sums_ref[1:2, :] / npart
    blk = x_ref[...]
    xb_ref[...] = jnp.sign(blk - m_full).astype(xb_ref.dtype)
    xtb_ref[...] = jnp.sign(blk - m_part)


def _layer0_body(aggA_ref, aggB_ref, cnt_ref, xtb_ref, wlA_ref, wlB_ref,
                 wr_ref, b_ref, h_ref, hsums_ref):
    i = pl.program_id(0)

    @pl.when(i == 0)
    def _():
        hsums_ref[...] = jnp.zeros_like(hsums_ref)

    cnt = jnp.maximum(cnt_ref[...], 1.0)
    aggA = aggA_ref[...].astype(jnp.float32)
    aggB = aggB_ref[...].astype(jnp.float32)
    hblk = (jnp.dot(aggA / cnt, wlA_ref[...],
                    preferred_element_type=jnp.float32)
            + jnp.dot(aggB / cnt, wlB_ref[...],
                      preferred_element_type=jnp.float32)
            + jnp.dot(xtb_ref[...], wr_ref[...], preferred_element_type=jnp.float32)
            + b_ref[...])
    hblk = jnp.maximum(hblk, 0.0)
    h_ref[...] = hblk
    hsums_ref[0:1, :] += jnp.sum(hblk, axis=0, keepdims=True)
    rows = i * _HBLK + lax.broadcasted_iota(jnp.int32, (_HBLK, 1), 0)
    hsums_ref[1:2, :] += jnp.sum(jnp.where(rows < _ND1, hblk, 0.0),
                                 axis=0, keepdims=True)


def _final_body(aggA_ref, aggB_ref, cntA_ref, cntB_ref, htb_ref,
                wl_ref, wr_ref, b_ref, out_ref):
    cnt = jnp.maximum(cntA_ref[...] + cntB_ref[...], 1.0)
    ma = (aggA_ref[...].astype(jnp.float32)
          + aggB_ref[...].astype(jnp.float32)) / cnt
    z = (jnp.dot(ma, wl_ref[...], preferred_element_type=jnp.float32)
         + jnp.dot(htb_ref[...], wr_ref[...], preferred_element_type=jnp.float32)
         + b_ref[...])
    z = z - jnp.max(z, axis=1, keepdims=True)
    z = z - jnp.log(jnp.sum(jnp.exp(z), axis=1, keepdims=True))
    out_ref[...] = z


_IB = 1024
_CPB = _IB // _CH
_CAP0 = _EPT0 + 2 * _CH


def _make_agg_body(ept, nch, half, garb, z_stripe, out_rows, out_tiles,
                   core_splits_edges, col_split=False):
    nb = ept // _IB
    assert nb % 2 == 0 or nb == 1
    assert nch == nb * _CPB
    assert _CPB >= _NRING
    cw = (max(z_stripe, out_rows) + 15) // 16 * 16

    def body(tab, src, dst, zr, agg_out, cnt_out, *sc):
        srcb = [sc[0], sc[1]]
        dstb = [sc[2], sc[3]]
        rows = list(sc[4:4 + _NRING])
        sidx = list(sc[4 + _NRING:4 + 2 * _NRING])
        gidx = list(sc[4 + 2 * _NRING:4 + 3 * _NRING])
        ones_v, cstage_v, agg_sh, cnt_sh = sc[4 + 3 * _NRING:8 + 3 * _NRING]
        gsem = list(sc[8 + 3 * _NRING:8 + 4 * _NRING])
        isem = list(sc[8 + 4 * _NRING:10 + 4 * _NRING])
        c = lax.axis_index("c")
        s = lax.axis_index("s")
        dbase = 0 if (core_splits_edges or col_split) else c * half
        ebase = (c * (ept * _NS) if core_splits_edges else 0) + s * ept

        def zbody(i, carry):
            cstage_v[pl.ds(i * 16, 16)] = jnp.zeros((16,), jnp.float32)
            return carry

        lax.fori_loop(0, cw // 16, zbody, 0)
        pltpu.sync_copy(zr.at[pl.ds(s * z_stripe, z_stripe)],
                        agg_sh.at[pl.ds(s * z_stripe, z_stripe)])
        pltpu.sync_copy(cstage_v.at[pl.ds(0, z_stripe)],
                        cnt_sh.at[pl.ds(s * z_stripe, z_stripe)])
        for k in range(_CH // 16):
            ones_v[pl.ds(k * 16, 16)] = jnp.full((16,), 1.0, jnp.float32)
        pltpu.sync_copy(src.at[pl.ds(ebase, _IB)], srcb[0])
        pltpu.sync_copy(dst.at[pl.ds(ebase, _IB)], dstb[0])
        if nb > 1:
            pltpu.async_copy(src.at[pl.ds(ebase + _IB, _IB)], srcb[1], isem[1])
            pltpu.async_copy(dst.at[pl.ds(ebase + _IB, _IB)], dstb[1], isem[1])
        plsc.subcore_barrier()

        garb_u = jnp.uint32(garb)

        def compute_sidx(dbuf, off, xbuf):
            for k in range(_CH // 16):
                d = dbuf[pl.ds(off + k * 16, 16)]
                loc = plsc.bitcast(d - dbase, jnp.uint32)
                xbuf[pl.ds(k * 16, 16)] = plsc.bitcast(
                    jnp.minimum(loc, garb_u), jnp.int32)

        def gather(sbuf, off, b):
            if not col_split:
                return pltpu.async_copy(
                    tab.at[sbuf.at[pl.ds(off, _CH)]], rows[b], gsem[b])
            for k in range(_CH // 16):
                sv = sbuf[pl.ds(off + k * 16, 16)]
                gidx[b][pl.ds(k * 16, 16)] = (sv << 1) + c
            return pltpu.async_copy(tab.at[gidx[b]], rows[b], gsem[b])

        def gather_wait(sbuf, off, b):
            if not col_split:
                pltpu.make_async_copy(
                    tab.at[sbuf.at[pl.ds(off, _CH)]], rows[b], gsem[b]).wait()
            else:
                pltpu.make_async_copy(tab.at[gidx[b]], rows[b], gsem[b]).wait()

        def scatter(b):
            pltpu.sync_copy(rows[b], agg_sh.at[sidx[b]], add=True)
            if col_split:
                @pl.when(c == 0)
                def _():
                    pltpu.sync_copy(ones_v, cnt_sh.at[sidx[b]], add=True)
            else:
                pltpu.sync_copy(ones_v, cnt_sh.at[sidx[b]], add=True)

        for b in range(_NRING):
            compute_sidx(dstb[0], b * _CH, sidx[b])
            gather(srcb[0], b * _CH, b)

        def pair_body(t, carry):
            for hm in range(2):
                m = 2 * t + hm
                bufm = hm
                bufn = 1 - hm
                for p in range(_CPB):
                    b = p % _NRING
                    gather_wait(srcb[bufm], p * _CH, b)
                    scatter(b)
                    if p == _CPB - _NRING:
                        @pl.when(m + 2 < nb)
                        def _():
                            off = ebase + (m + 2) * _IB
                            pltpu.async_copy(src.at[pl.ds(off, _IB)],
                                             srcb[bufm], isem[bufm])
                            pltpu.async_copy(dst.at[pl.ds(off, _IB)],
                                             dstb[bufm], isem[bufm])

                        @pl.when(m + 1 < nb)
                        def _():
                            off = ebase + (m + 1) * _IB
                            pltpu.make_async_copy(
                                src.at[pl.ds(off, _IB)], srcb[bufn],
                                isem[bufn]).wait()
                            pltpu.make_async_copy(
                                dst.at[pl.ds(off, _IB)], dstb[bufn],
                                isem[bufn]).wait()
                    j = m * _CPB + p + _NRING
                    if p < _CPB - _NRING:
                        jbuf, joff = bufm, (p + _NRING) * _CH
                    else:
                        jbuf, joff = bufn, (p + _NRING - _CPB) * _CH

                    @pl.when(j < nch)
                    def _():
                        compute_sidx(dstb[jbuf], joff, sidx[b])
                        gather(srcb[jbuf], joff, b)
            return carry

        lax.fori_loop(0, max(nb // 2, 1), pair_body, 0)
        plsc.subcore_barrier()

        @pl.when(s < out_tiles)
        def _():
            pltpu.sync_copy(agg_sh.at[pl.ds(s * out_rows, out_rows)],
                            agg_out.at[pl.ds(c * half + s * out_rows, out_rows)])

        cnt_base = (s * out_rows) if col_split else (c * half + s * out_rows)
        cnt_write = ((c == 0) & (s < out_tiles)) if col_split else (s < out_tiles)

        @pl.when(cnt_write)
        def _():
            pltpu.sync_copy(cnt_sh.at[pl.ds(s * out_rows, out_rows)],
                            cstage_v.at[pl.ds(0, out_rows)])
            pltpu.sync_copy(cstage_v.at[pl.ds(0, out_rows)],
                            cnt_out.at[pl.ds(cnt_base, out_rows)])

    return body


def _agg_scratch(rows, cstage, w, dtype=jnp.float32):
    return ([pltpu.VMEM((_IB,), jnp.int32)] * 2
            + [pltpu.VMEM((_IB,), jnp.int32)] * 2
            + [pltpu.VMEM((_CH, w), dtype)] * _NRING
            + [pltpu.VMEM((_CH,), jnp.int32)] * _NRING
            + [pltpu.VMEM((_CH,), jnp.int32)] * _NRING
            + [pltpu.VMEM((_CH,), jnp.float32),
               pltpu.VMEM((cstage,), jnp.float32),
               pltpu.VMEM_SHARED((rows, w), dtype),
               pltpu.VMEM_SHARED((rows,), jnp.float32)]
            + [pltpu.SemaphoreType.DMA] * (_NRING + 2))


_ROWS0C = 20480


@functools.cache
def _sc_kernels():
    mesh = plsc.VectorSubcoreMesh(core_axis_name="c", subcore_axis_name="s",
                                  num_cores=_NC, num_subcores=_NS)
    agg0 = pl.kernel(
        _make_agg_body(_EPT0, _NCH0, _ND0, _ROWS0C - 1, z_stripe=1280,
                       out_rows=2000, out_tiles=10, core_splits_edges=False,
                       col_split=True),
        out_type=[jax.ShapeDtypeStruct((_NC * _ND0, _IN_CH // 2), jnp.bfloat16),
                  jax.ShapeDtypeStruct((_ND0,), jnp.float32)],
        mesh=mesh,
        scratch_types=_agg_scratch(_ROWS0C, 2000, _IN_CH // 2, jnp.bfloat16),
        compiler_params=pltpu.CompilerParams(use_tc_tiling_on_sc=False),
    )
    agg1 = pl.kernel(
        _make_agg_body(_EPT1, _NCH1, _ND1, _ND1 - 1, z_stripe=256,
                       out_rows=256, out_tiles=_NS, core_splits_edges=True),
        out_type=[jax.ShapeDtypeStruct((_NC * _ND1, _HID), jnp.bfloat16),
                  jax.ShapeDtypeStruct((_NC * _ND1,), jnp.float32)],
        mesh=mesh,
        scratch_types=_agg_scratch(_ND1, 256, _HID, jnp.bfloat16),
        compiler_params=pltpu.CompilerParams(use_tc_tiling_on_sc=False),
    )
    return agg0, agg1


def kernel(x, edge_index0, edge_index1, size0_dst, size1_dst,
           W_l0, W_r0, b0, W_l1, W_r1, b1):
    f32 = jnp.float32
    x = x.astype(f32)
    src0 = edge_index0[0]
    dst0 = edge_index0[1]
    src1 = edge_index1[0]
    dst1 = edge_index1[1]

    sums = pl.pallas_call(
        _colsum_body,
        grid=(_N_SRC // _CBLK,),
        in_specs=[pl.BlockSpec((_CBLK, _IN_CH), lambda i: (i, 0))],
        out_specs=pl.BlockSpec((8, _IN_CH), lambda i: (0, 0)),
        out_shape=jax.ShapeDtypeStruct((8, _IN_CH), f32),
    )(x)

    xb, xtb = pl.pallas_call(
        functools.partial(_binarize_body, nfull=float(_N_SRC), npart=float(_ND0)),
        grid=(_ND0 // _BBLK,),
        in_specs=[pl.BlockSpec((8, _IN_CH), lambda i: (0, 0)),
                  pl.BlockSpec((_BBLK, _IN_CH), lambda i: (i, 0))],
        out_specs=[pl.BlockSpec((_BBLK, _IN_CH), lambda i: (i, 0))] * 2,
        out_shape=[jax.ShapeDtypeStruct((_ND0, _IN_CH), jnp.bfloat16),
                   jax.ShapeDtypeStruct((_ND0, _IN_CH), f32)],
    )(sums, x)

    zr0 = jnp.zeros((_ROWS0C, _IN_CH // 2), jnp.bfloat16)
    zr1 = jnp.zeros((_NS * 256, _HID), jnp.bfloat16)
    npad = _E0P - _E0
    src0p = jnp.concatenate([src0, jnp.zeros((npad,), jnp.int32)])
    dst0p = jnp.concatenate([dst0, jnp.full((npad,), _ND0, jnp.int32)])
    sc_agg0, sc_agg1 = _sc_kernels()
    agg0p, cnt0 = sc_agg0(xb.reshape(_NC * _ND0, _IN_CH // 2), src0p, dst0p,
                          zr0)

    h, hsums = pl.pallas_call(
        _layer0_body,
        grid=(_ND0 // _HBLK,),
        in_specs=[pl.BlockSpec((_HBLK, _IN_CH // 2), lambda i: (i, 0)),
                  pl.BlockSpec((_HBLK, _IN_CH // 2), lambda i: (i, 0)),
                  pl.BlockSpec((_HBLK, 1), lambda i: (i, 0)),
                  pl.BlockSpec((_HBLK, _IN_CH), lambda i: (i, 0)),
                  pl.BlockSpec((_IN_CH // 2, _HID), lambda i: (0, 0)),
                  pl.BlockSpec((_IN_CH // 2, _HID), lambda i: (0, 0)),
                  pl.BlockSpec((_IN_CH, _HID), lambda i: (0, 0)),
                  pl.BlockSpec((1, _HID), lambda i: (0, 0))],
        out_specs=[pl.BlockSpec((_HBLK, _HID), lambda i: (i, 0)),
                   pl.BlockSpec((8, _HID), lambda i: (0, 0))],
        out_shape=[jax.ShapeDtypeStruct((_ND0, _HID), f32),
                   jax.ShapeDtypeStruct((8, _HID), f32)],
    )(agg0p[:_ND0], agg0p[_ND0:], cnt0.reshape(_ND0, 1), xtb,
      W_l0[:_IN_CH // 2], W_l0[_IN_CH // 2:], W_r0, b0.reshape(1, _HID))

    hb, htb = pl.pallas_call(
        functools.partial(_binarize_body, nfull=float(_ND0), npart=float(_ND1)),
        grid=(_ND1 // _FBLK,),
        in_specs=[pl.BlockSpec((8, _HID), lambda i: (0, 0)),
                  pl.BlockSpec((_FBLK, _HID), lambda i: (i, 0))],
        out_specs=[pl.BlockSpec((_FBLK, _HID), lambda i: (i, 0))] * 2,
        out_shape=[jax.ShapeDtypeStruct((_ND1, _HID), jnp.bfloat16),
                   jax.ShapeDtypeStruct((_ND1, _HID), f32)],
    )(hsums, h)

    agg1p, cnt1p = sc_agg1(hb, src1, dst1, zr1)

    out = pl.pallas_call(
        _final_body,
        grid=(_ND1 // _FBLK,),
        in_specs=[pl.BlockSpec((_FBLK, _HID), lambda i: (i, 0)),
                  pl.BlockSpec((_FBLK, _HID), lambda i: (i, 0)),
                  pl.BlockSpec((_FBLK, 1), lambda i: (i, 0)),
                  pl.BlockSpec((_FBLK, 1), lambda i: (i, 0)),
                  pl.BlockSpec((_FBLK, _HID), lambda i: (i, 0)),
                  pl.BlockSpec((_HID, _OUT), lambda i: (0, 0)),
                  pl.BlockSpec((_HID, _OUT), lambda i: (0, 0)),
                  pl.BlockSpec((1, _OUT), lambda i: (0, 0))],
        out_specs=pl.BlockSpec((_FBLK, _OUT), lambda i: (i, 0)),
        out_shape=jax.ShapeDtypeStruct((_ND1, _OUT), f32),
    )(agg1p[:_ND1], agg1p[_ND1:], cnt1p[:_ND1].reshape(_ND1, 1),
      cnt1p[_ND1:].reshape(_ND1, 1), htb, W_l1, W_r1, b1.reshape(1, _OUT))
    return out


# speedup vs baseline: 2.2946x; 1.0070x over previous
import functools

import jax
import jax.numpy as jnp
from jax import lax
from jax.experimental import pallas as pl
from jax.experimental.pallas import tpu as pltpu
from jax.experimental.pallas import tpu_sc as plsc

_IN_CH = 128
_HID = 128
_OUT = 64
_N_SRC = 100000
_ND0 = 20000
_ND1 = 4096
_E0 = 320000
_E1 = 65536
_NC = 2
_NS = 16

_CH = 128
_NRING = 8

_HALF0 = _ND0 // 2
_ROWS0 = 10240
_GARB0 = _HALF0
_EPT0 = 20480
_E0P = _EPT0 * _NS
_NCH0 = _EPT0 // _CH

_EPC1 = _E1 // _NC
_EPT1 = _EPC1 // _NS
_NCH1 = _EPT1 // _CH

_CBLK = 2000
_BBLK = 2000
_HBLK = 400
_FBLK = 512


def _colsum_body(x_ref, out_ref):
    i = pl.program_id(0)

    @pl.when(i == 0)
    def _():
        out_ref[...] = jnp.zeros_like(out_ref)

    ssum = jnp.sum(x_ref[...], axis=0, keepdims=True)
    out_ref[0:1, :] += ssum

    @pl.when(i < _ND0 // _CBLK)
    def _():
        out_ref[1:2, :] += ssum


def _binarize_body(sums_ref, x_ref, xb_ref, xtb_ref, *, nfull, npart):
    m_full = sums_ref[0:1, :] / nfull
    m_part = sums_ref[1:2, :] / npart
    blk = x_ref[...]
    xb_ref[...] = jnp.sign(blk - m_full).astype(xb_ref.dtype)
    xtb_ref[...] = jnp.sign(blk - m_part)


def _layer0_body(aggA_ref, aggB_ref, cnt_ref, xtb_ref, wlA_ref, wlB_ref,
                 wr_ref, b_ref, h_ref, hsums_ref):
    i = pl.program_id(0)

    @pl.when(i == 0)
    def _():
        hsums_ref[...] = jnp.zeros_like(hsums_ref)

    cnt = jnp.maximum(cnt_ref[...], 1.0)
    aggA = aggA_ref[...].astype(jnp.float32)
    aggB = aggB_ref[...].astype(jnp.float32)
    hblk = (jnp.dot(aggA / cnt, wlA_ref[...],
                    preferred_element_type=jnp.float32)
            + jnp.dot(aggB / cnt, wlB_ref[...],
                      preferred_element_type=jnp.float32)
            + jnp.dot(xtb_ref[...], wr_ref[...], preferred_element_type=jnp.float32)
            + b_ref[...])
    hblk = jnp.maximum(hblk, 0.0)
    h_ref[...] = hblk
    hsums_ref[0:1, :] += jnp.sum(hblk, axis=0, keepdims=True)
    rows = i * _HBLK + lax.broadcasted_iota(jnp.int32, (_HBLK, 1), 0)
    hsums_ref[1:2, :] += jnp.sum(jnp.where(rows < _ND1, hblk, 0.0),
                                 axis=0, keepdims=True)


def _final_body(aggA_ref, aggB_ref, cntA_ref, cntB_ref, htb_ref,
                wl_ref, wr_ref, b_ref, out_ref):
    cnt = jnp.maximum(cntA_ref[...] + cntB_ref[...], 1.0)
    ma = (aggA_ref[...].astype(jnp.float32)
          + aggB_ref[...].astype(jnp.float32)) / cnt
    z = (jnp.dot(ma, wl_ref[...], preferred_element_type=jnp.float32)
         + jnp.dot(htb_ref[...], wr_ref[...], preferred_element_type=jnp.float32)
         + b_ref[...])
    z = z - jnp.max(z, axis=1, keepdims=True)
    z = z - jnp.log(jnp.sum(jnp.exp(z), axis=1, keepdims=True))
    out_ref[...] = z


_IB = 1024
_CPB = _IB // _CH
_CAP0 = _EPT0 + 2 * _CH


def _make_agg_body(ept, nch, half, garb, z_stripe, out_rows, out_tiles,
                   core_splits_edges, col_split=False):
    nb = ept // _IB
    assert nb % 2 == 0 or nb == 1
    assert nch == nb * _CPB
    assert _CPB >= _NRING
    cw = (max(z_stripe, out_rows) + 15) // 16 * 16

    def body(tab, src, dst, zr, agg_out, cnt_out, *sc):
        srcb = [sc[0], sc[1]]
        dstb = [sc[2], sc[3]]
        rows = list(sc[4:4 + _NRING])
        sidx = list(sc[4 + _NRING:4 + 2 * _NRING])
        gidx = list(sc[4 + 2 * _NRING:4 + 3 * _NRING])
        ones_v, cstage_v, agg_sh, cnt_sh = sc[4 + 3 * _NRING:8 + 3 * _NRING]
        gsem = list(sc[8 + 3 * _NRING:8 + 4 * _NRING])
        isem = list(sc[8 + 4 * _NRING:10 + 4 * _NRING])
        c = lax.axis_index("c")
        s = lax.axis_index("s")
        dbase = 0 if (core_splits_edges or col_split) else c * half
        ebase = (c * (ept * _NS) if core_splits_edges else 0) + s * ept

        def zbody(i, carry):
            cstage_v[pl.ds(i * 16, 16)] = jnp.zeros((16,), jnp.float32)
            return carry

        lax.fori_loop(0, cw // 16, zbody, 0)
        pltpu.sync_copy(zr.at[pl.ds(s * z_stripe, z_stripe)],
                        agg_sh.at[pl.ds(s * z_stripe, z_stripe)])
        pltpu.sync_copy(cstage_v.at[pl.ds(0, z_stripe)],
                        cnt_sh.at[pl.ds(s * z_stripe, z_stripe)])
        for k in range(_CH // 16):
            ones_v[pl.ds(k * 16, 16)] = jnp.full((16,), 1.0, jnp.float32)
        pltpu.sync_copy(src.at[pl.ds(ebase, _IB)], srcb[0])
        pltpu.sync_copy(dst.at[pl.ds(ebase, _IB)], dstb[0])
        if nb > 1:
            pltpu.async_copy(src.at[pl.ds(ebase + _IB, _IB)], srcb[1], isem[1])
            pltpu.async_copy(dst.at[pl.ds(ebase + _IB, _IB)], dstb[1], isem[1])
        plsc.subcore_barrier()

        garb_u = jnp.uint32(garb)

        def compute_sidx(dbuf, off, xbuf):
            for k in range(_CH // 16):
                d = dbuf[pl.ds(off + k * 16, 16)]
                loc = plsc.bitcast(d - dbase, jnp.uint32)
                xbuf[pl.ds(k * 16, 16)] = plsc.bitcast(
                    jnp.minimum(loc, garb_u), jnp.int32)

        def gather(sbuf, off, b):
            if not col_split:
                return pltpu.async_copy(
                    tab.at[sbuf.at[pl.ds(off, _CH)]], rows[b], gsem[b])
            for k in range(_CH // 16):
                sv = sbuf[pl.ds(off + k * 16, 16)]
                gidx[b][pl.ds(k * 16, 16)] = (sv << 1) + c
            return pltpu.async_copy(tab.at[gidx[b]], rows[b], gsem[b])

        def gather_wait(sbuf, off, b):
            if not col_split:
                pltpu.make_async_copy(
                    tab.at[sbuf.at[pl.ds(off, _CH)]], rows[b], gsem[b]).wait()
            else:
                pltpu.make_async_copy(tab.at[gidx[b]], rows[b], gsem[b]).wait()

        def scatter(b):
            pltpu.sync_copy(rows[b], agg_sh.at[sidx[b]], add=True)
            if col_split:
                @pl.when(c == 0)
                def _():
                    pltpu.sync_copy(ones_v, cnt_sh.at[sidx[b]], add=True)
            else:
                pltpu.sync_copy(ones_v, cnt_sh.at[sidx[b]], add=True)

        for b in range(_NRING):
            compute_sidx(dstb[0], b * _CH, sidx[b])
            gather(srcb[0], b * _CH, b)

        def pair_body(t, carry):
            for hm in range(2):
                m = 2 * t + hm
                bufm = hm
                bufn = 1 - hm
                for p in range(_CPB):
                    b = p % _NRING
                    gather_wait(srcb[bufm], p * _CH, b)
                    scatter(b)
                    if p == _CPB - _NRING:
                        @pl.when(m + 2 < nb)
                        def _():
                            off = ebase + (m + 2) * _IB
                            pltpu.async_copy(src.at[pl.ds(off, _IB)],
                                             srcb[bufm], isem[bufm])
                            pltpu.async_copy(dst.at[pl.ds(off, _IB)],
                                             dstb[bufm], isem[bufm])

                        @pl.when(m + 1 < nb)
                        def _():
                            off = ebase + (m + 1) * _IB
                            pltpu.make_async_copy(
                                src.at[pl.ds(off, _IB)], srcb[bufn],
                                isem[bufn]).wait()
                            pltpu.make_async_copy(
                                dst.at[pl.ds(off, _IB)], dstb[bufn],
                                isem[bufn]).wait()
                    j = m * _CPB + p + _NRING
                    if p < _CPB - _NRING:
                        jbuf, joff = bufm, (p + _NRING) * _CH
                    else:
                        jbuf, joff = bufn, (p + _NRING - _CPB) * _CH

                    @pl.when(j < nch)
                    def _():
                        compute_sidx(dstb[jbuf], joff, sidx[b])
                        gather(srcb[jbuf], joff, b)
            return carry

        lax.fori_loop(0, max(nb // 2, 1), pair_body, 0)
        plsc.subcore_barrier()

        @pl.when(s < out_tiles)
        def _():
            pltpu.sync_copy(agg_sh.at[pl.ds(s * out_rows, out_rows)],
                            agg_out.at[pl.ds(c * half + s * out_rows, out_rows)])

        cnt_base = (s * out_rows) if col_split else (c * half + s * out_rows)
        cnt_write = ((c == 0) & (s < out_tiles)) if col_split else (s < out_tiles)

        @pl.when(cnt_write)
        def _():
            pltpu.sync_copy(cnt_sh.at[pl.ds(s * out_rows, out_rows)],
                            cstage_v.at[pl.ds(0, out_rows)])
            pltpu.sync_copy(cstage_v.at[pl.ds(0, out_rows)],
                            cnt_out.at[pl.ds(cnt_base, out_rows)])

    return body


def _agg_scratch(rows, cstage, w, dtype=jnp.float32):
    return ([pltpu.VMEM((_IB,), jnp.int32)] * 2
            + [pltpu.VMEM((_IB,), jnp.int32)] * 2
            + [pltpu.VMEM((_CH, w), dtype)] * _NRING
            + [pltpu.VMEM((_CH,), jnp.int32)] * _NRING
            + [pltpu.VMEM((_CH,), jnp.int32)] * _NRING
            + [pltpu.VMEM((_CH,), jnp.float32),
               pltpu.VMEM((cstage,), jnp.float32),
               pltpu.VMEM_SHARED((rows, w), dtype),
               pltpu.VMEM_SHARED((rows,), jnp.float32)]
            + [pltpu.SemaphoreType.DMA] * (_NRING + 2))


_ROWS0C = 20480


@functools.cache
def _sc_kernels():
    mesh = plsc.VectorSubcoreMesh(core_axis_name="c", subcore_axis_name="s",
                                  num_cores=_NC, num_subcores=_NS)
    agg0 = pl.kernel(
        _make_agg_body(_EPT0, _NCH0, _ND0, _ROWS0C - 1, z_stripe=1280,
                       out_rows=2000, out_tiles=10, core_splits_edges=False,
                       col_split=True),
        out_type=[jax.ShapeDtypeStruct((_NC * _ND0, _IN_CH // 2), jnp.bfloat16),
                  jax.ShapeDtypeStruct((_ND0,), jnp.float32)],
        mesh=mesh,
        scratch_types=_agg_scratch(_ROWS0C, 2000, _IN_CH // 2, jnp.bfloat16),
        compiler_params=pltpu.CompilerParams(use_tc_tiling_on_sc=False),
    )
    agg1 = pl.kernel(
        _make_agg_body(_EPT1, _NCH1, _ND1, _ND1 - 1, z_stripe=256,
                       out_rows=256, out_tiles=_NS, core_splits_edges=True),
        out_type=[jax.ShapeDtypeStruct((_NC * _ND1, _HID), jnp.bfloat16),
                  jax.ShapeDtypeStruct((_NC * _ND1,), jnp.float32)],
        mesh=mesh,
        scratch_types=_agg_scratch(_ND1, 256, _HID, jnp.bfloat16),
        compiler_params=pltpu.CompilerParams(use_tc_tiling_on_sc=False),
    )
    return agg0, agg1


def kernel(x, edge_index0, edge_index1, size0_dst, size1_dst,
           W_l0, W_r0, b0, W_l1, W_r1, b1):
    f32 = jnp.float32
    x = x.astype(f32)
    src0 = edge_index0[0]
    dst0 = edge_index0[1]
    src1 = edge_index1[0]
    dst1 = edge_index1[1]

    sums = pl.pallas_call(
        _colsum_body,
        grid=(_N_SRC // _CBLK,),
        in_specs=[pl.BlockSpec((_CBLK, _IN_CH), lambda i: (i, 0))],
        out_specs=pl.BlockSpec((8, _IN_CH), lambda i: (0, 0)),
        out_shape=jax.ShapeDtypeStruct((8, _IN_CH), f32),
    )(x)

    xb, xtb = pl.pallas_call(
        functools.partial(_binarize_body, nfull=float(_N_SRC), npart=float(_ND0)),
        grid=(_ND0 // _BBLK,),
        in_specs=[pl.BlockSpec((8, _IN_CH), lambda i: (0, 0)),
                  pl.BlockSpec((_BBLK, _IN_CH), lambda i: (i, 0))],
        out_specs=[pl.BlockSpec((_BBLK, _IN_CH), lambda i: (i, 0))] * 2,
        out_shape=[jax.ShapeDtypeStruct((_ND0, _IN_CH), jnp.bfloat16),
                   jax.ShapeDtypeStruct((_ND0, _IN_CH), f32)],
    )(sums, x)

    zr0 = jnp.zeros((_ROWS0C, _IN_CH // 2), jnp.bfloat16)
    zr1 = jnp.zeros((_NS * 256, _HID), jnp.bfloat16)
    npad = _E0P - _E0
    src0p = jnp.concatenate([src0, jnp.zeros((npad,), jnp.int32)])
    dst0p = jnp.concatenate([dst0, jnp.full((npad,), _ND0, jnp.int32)])
    sc_agg0, sc_agg1 = _sc_kernels()
    agg0p, cnt0 = sc_agg0(xb.reshape(_NC * _ND0, _IN_CH // 2), src0p, dst0p,
                          zr0)

    h, hsums = pl.pallas_call(
        _layer0_body,
        grid=(_ND0 // _HBLK,),
        in_specs=[pl.BlockSpec((_HBLK, _IN_CH // 2), lambda i: (i, 0)),
                  pl.BlockSpec((_HBLK, _IN_CH // 2), lambda i: (i, 0)),
                  pl.BlockSpec((_HBLK, 1), lambda i: (i, 0)),
                  pl.BlockSpec((_HBLK, _IN_CH), lambda i: (i, 0)),
                  pl.BlockSpec((_IN_CH // 2, _HID), lambda i: (0, 0)),
                  pl.BlockSpec((_IN_CH // 2, _HID), lambda i: (0, 0)),
                  pl.BlockSpec((_IN_CH, _HID), lambda i: (0, 0)),
                  pl.BlockSpec((1, _HID), lambda i: (0, 0))],
        out_specs=[pl.BlockSpec((_HBLK, _HID), lambda i: (i, 0)),
                   pl.BlockSpec((8, _HID), lambda i: (0, 0))],
        out_shape=[jax.ShapeDtypeStruct((_ND0, _HID), f32),
                   jax.ShapeDtypeStruct((8, _HID), f32)],
    )(agg0p[:_ND0], agg0p[_ND0:], cnt0.reshape(_ND0, 1), xtb,
      W_l0[:_IN_CH // 2], W_l0[_IN_CH // 2:], W_r0, b0.reshape(1, _HID))

    hb, htb = pl.pallas_call(
        functools.partial(_binarize_body, nfull=float(_ND0), npart=float(_ND1)),
        grid=(_ND1 // _FBLK,),
        in_specs=[pl.BlockSpec((8, _HID), lambda i: (0, 0)),
                  pl.BlockSpec((_FBLK, _HID), lambda i: (i, 0))],
        out_specs=[pl.BlockSpec((_FBLK, _HID), lambda i: (i, 0))] * 2,
        out_shape=[jax.ShapeDtypeStruct((_ND1, _HID), jnp.bfloat16),
                   jax.ShapeDtypeStruct((_ND1, _HID), f32)],
    )(hsums, h)

    agg1p, cnt1p = sc_agg1(hb, src1, dst1, zr1)

    out = pl.pallas_call(
        _final_body,
        grid=(_ND1 // _FBLK,),
        in_specs=[pl.BlockSpec((_FBLK, _HID), lambda i: (i, 0)),
                  pl.BlockSpec((_FBLK, _HID), lambda i: (i, 0)),
                  pl.BlockSpec((_FBLK, 1), lambda i: (i, 0)),
                  pl.BlockSpec((_FBLK, 1), lambda i: (i, 0)),
                  pl.BlockSpec((_FBLK, _HID), lambda i: (i, 0)),
                  pl.BlockSpec((_HID, _OUT), lambda i: (0, 0)),
                  pl.BlockSpec((_HID, _OUT), lambda i: (0, 0)),
                  pl.BlockSpec((1, _OUT), lambda i: (0, 0))],
        out_specs=pl.BlockSpec((_FBLK, _OUT), lambda i: (i, 0)),
        out_shape=jax.ShapeDtypeStruct((_ND1, _OUT), f32),
    )(agg1p[:_ND1], agg1p[_ND1:], cnt1p[:_ND1].reshape(_ND1, 1),
      cnt1p[_ND1:].reshape(_ND1, 1), htb, W_l1, W_r1, b1.reshape(1, _OUT))
    return out


# speedup vs baseline: 2.3034x; 1.0038x over previous
import functools

import jax
import jax.numpy as jnp
from jax import lax
from jax.experimental import pallas as pl
from jax.experimental.pallas import tpu as pltpu
from jax.experimental.pallas import tpu_sc as plsc

_IN_CH = 128
_HID = 128
_OUT = 64
_N_SRC = 100000
_ND0 = 20000
_ND1 = 4096
_E0 = 320000
_E1 = 65536
_NC = 2
_NS = 16

_CH = 128
_NRING = 8

_HALF0 = _ND0 // 2
_ROWS0 = 10240
_GARB0 = _HALF0
_EPT0 = 20480
_E0P = _EPT0 * _NS
_NCH0 = _EPT0 // _CH

_EPC1 = _E1 // _NC
_EPT1 = _EPC1 // _NS
_NCH1 = _EPT1 // _CH

_CBLK = 2000
_BBLK = 2000
_HBLK = 400
_FBLK = 512


def _colsum_body(x_ref, out_ref):
    i = pl.program_id(0)

    @pl.when(i == 0)
    def _():
        out_ref[...] = jnp.zeros_like(out_ref)

    ssum = jnp.sum(x_ref[...], axis=0, keepdims=True)
    out_ref[0:1, :] += ssum

    @pl.when(i < _ND0 // _CBLK)
    def _():
        out_ref[1:2, :] += ssum


def _binarize_body(sums_ref, x_ref, xb_ref, xtb_ref, *, nfull, npart):
    m_full = sums_ref[0:1, :] / nfull
    m_part = sums_ref[1:2, :] / npart
    blk = x_ref[...]
    xb_ref[...] = jnp.sign(blk - m_full).astype(xb_ref.dtype)
    xtb_ref[...] = jnp.sign(blk - m_part).astype(xtb_ref.dtype)


def _layer0_body(aggA_ref, aggB_ref, cnt_ref, xtb_ref, wlA_ref, wlB_ref,
                 wr_ref, b_ref, h_ref, hsums_ref):
    i = pl.program_id(0)

    @pl.when(i == 0)
    def _():
        hsums_ref[...] = jnp.zeros_like(hsums_ref)

    cnt = jnp.maximum(cnt_ref[...], 1.0)
    aggA = aggA_ref[...].astype(jnp.float32)
    aggB = aggB_ref[...].astype(jnp.float32)
    hblk = (jnp.dot(aggA / cnt, wlA_ref[...],
                    preferred_element_type=jnp.float32)
            + jnp.dot(aggB / cnt, wlB_ref[...],
                      preferred_element_type=jnp.float32)
            + jnp.dot(xtb_ref[...].astype(jnp.float32), wr_ref[...],
                      preferred_element_type=jnp.float32)
            + b_ref[...])
    hblk = jnp.maximum(hblk, 0.0)
    h_ref[...] = hblk
    hsums_ref[0:1, :] += jnp.sum(hblk, axis=0, keepdims=True)
    rows = i * _HBLK + lax.broadcasted_iota(jnp.int32, (_HBLK, 1), 0)
    hsums_ref[1:2, :] += jnp.sum(jnp.where(rows < _ND1, hblk, 0.0),
                                 axis=0, keepdims=True)


def _final_body(aggA_ref, aggB_ref, cntA_ref, cntB_ref, htb_ref,
                wl_ref, wr_ref, b_ref, out_ref):
    cnt = jnp.maximum(cntA_ref[...] + cntB_ref[...], 1.0)
    ma = (aggA_ref[...].astype(jnp.float32)
          + aggB_ref[...].astype(jnp.float32)) / cnt
    z = (jnp.dot(ma, wl_ref[...], preferred_element_type=jnp.float32)
         + jnp.dot(htb_ref[...].astype(jnp.float32), wr_ref[...],
                   preferred_element_type=jnp.float32)
         + b_ref[...])
    z = z - jnp.max(z, axis=1, keepdims=True)
    z = z - jnp.log(jnp.sum(jnp.exp(z), axis=1, keepdims=True))
    out_ref[...] = z


_IB = 1024
_CPB = _IB // _CH
_CAP0 = _EPT0 + 2 * _CH


def _make_agg_body(ept, nch, half, garb, z_stripe, out_rows, out_tiles,
                   core_splits_edges, col_split=False):
    nb = ept // _IB
    assert nb % 2 == 0 or nb == 1
    assert nch == nb * _CPB
    assert _CPB >= _NRING
    cw = (max(z_stripe, out_rows) + 15) // 16 * 16

    def body(tab, src, dst, zr, agg_out, cnt_out, *sc):
        srcb = [sc[0], sc[1]]
        dstb = [sc[2], sc[3]]
        rows = list(sc[4:4 + _NRING])
        sidx = list(sc[4 + _NRING:4 + 2 * _NRING])
        gidx = list(sc[4 + 2 * _NRING:4 + 3 * _NRING])
        ones_v, cstage_v, agg_sh, cnt_sh = sc[4 + 3 * _NRING:8 + 3 * _NRING]
        gsem = list(sc[8 + 3 * _NRING:8 + 4 * _NRING])
        isem = list(sc[8 + 4 * _NRING:10 + 4 * _NRING])
        c = lax.axis_index("c")
        s = lax.axis_index("s")
        dbase = 0 if (core_splits_edges or col_split) else c * half
        ebase = (c * (ept * _NS) if core_splits_edges else 0) + s * ept

        def zbody(i, carry):
            cstage_v[pl.ds(i * 16, 16)] = jnp.zeros((16,), jnp.float32)
            return carry

        lax.fori_loop(0, cw // 16, zbody, 0)
        pltpu.sync_copy(zr.at[pl.ds(s * z_stripe, z_stripe)],
                        agg_sh.at[pl.ds(s * z_stripe, z_stripe)])
        pltpu.sync_copy(cstage_v.at[pl.ds(0, z_stripe)],
                        cnt_sh.at[pl.ds(s * z_stripe, z_stripe)])
        for k in range(_CH // 16):
            ones_v[pl.ds(k * 16, 16)] = jnp.full((16,), 1.0, jnp.float32)
        pltpu.sync_copy(src.at[pl.ds(ebase, _IB)], srcb[0])
        pltpu.sync_copy(dst.at[pl.ds(ebase, _IB)], dstb[0])
        if nb > 1:
            pltpu.async_copy(src.at[pl.ds(ebase + _IB, _IB)], srcb[1], isem[1])
            pltpu.async_copy(dst.at[pl.ds(ebase + _IB, _IB)], dstb[1], isem[1])
        plsc.subcore_barrier()

        garb_u = jnp.uint32(garb)

        def compute_sidx(dbuf, off, xbuf):
            for k in range(_CH // 16):
                d = dbuf[pl.ds(off + k * 16, 16)]
                loc = plsc.bitcast(d - dbase, jnp.uint32)
                xbuf[pl.ds(k * 16, 16)] = plsc.bitcast(
                    jnp.minimum(loc, garb_u), jnp.int32)

        def gather(sbuf, off, b):
            if not col_split:
                return pltpu.async_copy(
                    tab.at[sbuf.at[pl.ds(off, _CH)]], rows[b], gsem[b])
            for k in range(_CH // 16):
                sv = sbuf[pl.ds(off + k * 16, 16)]
                gidx[b][pl.ds(k * 16, 16)] = (sv << 1) + c
            return pltpu.async_copy(tab.at[gidx[b]], rows[b], gsem[b])

        def gather_wait(sbuf, off, b):
            if not col_split:
                pltpu.make_async_copy(
                    tab.at[sbuf.at[pl.ds(off, _CH)]], rows[b], gsem[b]).wait()
            else:
                pltpu.make_async_copy(tab.at[gidx[b]], rows[b], gsem[b]).wait()

        def scatter(b):
            pltpu.sync_copy(rows[b], agg_sh.at[sidx[b]], add=True)
            if col_split:
                @pl.when(c == 0)
                def _():
                    pltpu.sync_copy(ones_v, cnt_sh.at[sidx[b]], add=True)
            else:
                pltpu.sync_copy(ones_v, cnt_sh.at[sidx[b]], add=True)

        for b in range(_NRING):
            compute_sidx(dstb[0], b * _CH, sidx[b])
            gather(srcb[0], b * _CH, b)

        def pair_body(t, carry):
            for hm in range(2):
                m = 2 * t + hm
                bufm = hm
                bufn = 1 - hm
                for p in range(_CPB):
                    b = p % _NRING
                    gather_wait(srcb[bufm], p * _CH, b)
                    scatter(b)
                    if p == _CPB - _NRING:
                        @pl.when(m + 2 < nb)
                        def _():
                            off = ebase + (m + 2) * _IB
                            pltpu.async_copy(src.at[pl.ds(off, _IB)],
                                             srcb[bufm], isem[bufm])
                            pltpu.async_copy(dst.at[pl.ds(off, _IB)],
                                             dstb[bufm], isem[bufm])

                        @pl.when(m + 1 < nb)
                        def _():
                            off = ebase + (m + 1) * _IB
                            pltpu.make_async_copy(
                                src.at[pl.ds(off, _IB)], srcb[bufn],
                                isem[bufn]).wait()
                            pltpu.make_async_copy(
                                dst.at[pl.ds(off, _IB)], dstb[bufn],
                                isem[bufn]).wait()
                    j = m * _CPB + p + _NRING
                    if p < _CPB - _NRING:
                        jbuf, joff = bufm, (p + _NRING) * _CH
                    else:
                        jbuf, joff = bufn, (p + _NRING - _CPB) * _CH

                    @pl.when(j < nch)
                    def _():
                        compute_sidx(dstb[jbuf], joff, sidx[b])
                        gather(srcb[jbuf], joff, b)
            return carry

        lax.fori_loop(0, max(nb // 2, 1), pair_body, 0)
        plsc.subcore_barrier()

        @pl.when(s < out_tiles)
        def _():
            pltpu.sync_copy(agg_sh.at[pl.ds(s * out_rows, out_rows)],
                            agg_out.at[pl.ds(c * half + s * out_rows, out_rows)])

        cnt_base = (s * out_rows) if col_split else (c * half + s * out_rows)
        cnt_write = ((c == 0) & (s < out_tiles)) if col_split else (s < out_tiles)

        @pl.when(cnt_write)
        def _():
            pltpu.sync_copy(cnt_sh.at[pl.ds(s * out_rows, out_rows)],
                            cstage_v.at[pl.ds(0, out_rows)])
            pltpu.sync_copy(cstage_v.at[pl.ds(0, out_rows)],
                            cnt_out.at[pl.ds(cnt_base, out_rows)])

    return body


def _agg_scratch(rows, cstage, w, dtype=jnp.float32):
    return ([pltpu.VMEM((_IB,), jnp.int32)] * 2
            + [pltpu.VMEM((_IB,), jnp.int32)] * 2
            + [pltpu.VMEM((_CH, w), dtype)] * _NRING
            + [pltpu.VMEM((_CH,), jnp.int32)] * _NRING
            + [pltpu.VMEM((_CH,), jnp.int32)] * _NRING
            + [pltpu.VMEM((_CH,), jnp.float32),
               pltpu.VMEM((cstage,), jnp.float32),
               pltpu.VMEM_SHARED((rows, w), dtype),
               pltpu.VMEM_SHARED((rows,), jnp.float32)]
            + [pltpu.SemaphoreType.DMA] * (_NRING + 2))


_ROWS0C = 20480


@functools.cache
def _sc_kernels():
    mesh = plsc.VectorSubcoreMesh(core_axis_name="c", subcore_axis_name="s",
                                  num_cores=_NC, num_subcores=_NS)
    agg0 = pl.kernel(
        _make_agg_body(_EPT0, _NCH0, _ND0, _ROWS0C - 1, z_stripe=1280,
                       out_rows=2000, out_tiles=10, core_splits_edges=False,
                       col_split=True),
        out_type=[jax.ShapeDtypeStruct((_NC * _ND0, _IN_CH // 2), jnp.bfloat16),
                  jax.ShapeDtypeStruct((_ND0,), jnp.float32)],
        mesh=mesh,
        scratch_types=_agg_scratch(_ROWS0C, 2000, _IN_CH // 2, jnp.bfloat16),
        compiler_params=pltpu.CompilerParams(use_tc_tiling_on_sc=False),
    )
    agg1 = pl.kernel(
        _make_agg_body(_EPT1, _NCH1, _ND1, _ND1 - 1, z_stripe=256,
                       out_rows=256, out_tiles=_NS, core_splits_edges=True),
        out_type=[jax.ShapeDtypeStruct((_NC * _ND1, _HID), jnp.bfloat16),
                  jax.ShapeDtypeStruct((_NC * _ND1,), jnp.float32)],
        mesh=mesh,
        scratch_types=_agg_scratch(_ND1, 256, _HID, jnp.bfloat16),
        compiler_params=pltpu.CompilerParams(use_tc_tiling_on_sc=False),
    )
    return agg0, agg1


def kernel(x, edge_index0, edge_index1, size0_dst, size1_dst,
           W_l0, W_r0, b0, W_l1, W_r1, b1):
    f32 = jnp.float32
    x = x.astype(f32)
    src0 = edge_index0[0]
    dst0 = edge_index0[1]
    src1 = edge_index1[0]
    dst1 = edge_index1[1]

    sums = pl.pallas_call(
        _colsum_body,
        grid=(_N_SRC // _CBLK,),
        in_specs=[pl.BlockSpec((_CBLK, _IN_CH), lambda i: (i, 0))],
        out_specs=pl.BlockSpec((8, _IN_CH), lambda i: (0, 0)),
        out_shape=jax.ShapeDtypeStruct((8, _IN_CH), f32),
    )(x)

    xb, xtb = pl.pallas_call(
        functools.partial(_binarize_body, nfull=float(_N_SRC), npart=float(_ND0)),
        grid=(_ND0 // _BBLK,),
        in_specs=[pl.BlockSpec((8, _IN_CH), lambda i: (0, 0)),
                  pl.BlockSpec((_BBLK, _IN_CH), lambda i: (i, 0))],
        out_specs=[pl.BlockSpec((_BBLK, _IN_CH), lambda i: (i, 0))] * 2,
        out_shape=[jax.ShapeDtypeStruct((_ND0, _IN_CH), jnp.bfloat16)] * 2,
    )(sums, x)

    zr0 = jnp.zeros((_ROWS0C, _IN_CH // 2), jnp.bfloat16)
    zr1 = jnp.zeros((_NS * 256, _HID), jnp.bfloat16)
    npad = _E0P - _E0
    src0p = jnp.concatenate([src0, jnp.zeros((npad,), jnp.int32)])
    dst0p = jnp.concatenate([dst0, jnp.full((npad,), _ND0, jnp.int32)])
    sc_agg0, sc_agg1 = _sc_kernels()
    agg0p, cnt0 = sc_agg0(xb.reshape(_NC * _ND0, _IN_CH // 2), src0p, dst0p,
                          zr0)

    h, hsums = pl.pallas_call(
        _layer0_body,
        grid=(_ND0 // _HBLK,),
        in_specs=[pl.BlockSpec((_HBLK, _IN_CH // 2), lambda i: (i, 0)),
                  pl.BlockSpec((_HBLK, _IN_CH // 2), lambda i: (i, 0)),
                  pl.BlockSpec((_HBLK, 1), lambda i: (i, 0)),
                  pl.BlockSpec((_HBLK, _IN_CH), lambda i: (i, 0)),
                  pl.BlockSpec((_IN_CH // 2, _HID), lambda i: (0, 0)),
                  pl.BlockSpec((_IN_CH // 2, _HID), lambda i: (0, 0)),
                  pl.BlockSpec((_IN_CH, _HID), lambda i: (0, 0)),
                  pl.BlockSpec((1, _HID), lambda i: (0, 0))],
        out_specs=[pl.BlockSpec((_HBLK, _HID), lambda i: (i, 0)),
                   pl.BlockSpec((8, _HID), lambda i: (0, 0))],
        out_shape=[jax.ShapeDtypeStruct((_ND0, _HID), f32),
                   jax.ShapeDtypeStruct((8, _HID), f32)],
    )(agg0p[:_ND0], agg0p[_ND0:], cnt0.reshape(_ND0, 1), xtb,
      W_l0[:_IN_CH // 2], W_l0[_IN_CH // 2:], W_r0, b0.reshape(1, _HID))

    hb, htb = pl.pallas_call(
        functools.partial(_binarize_body, nfull=float(_ND0), npart=float(_ND1)),
        grid=(_ND1 // _FBLK,),
        in_specs=[pl.BlockSpec((8, _HID), lambda i: (0, 0)),
                  pl.BlockSpec((_FBLK, _HID), lambda i: (i, 0))],
        out_specs=[pl.BlockSpec((_FBLK, _HID), lambda i: (i, 0))] * 2,
        out_shape=[jax.ShapeDtypeStruct((_ND1, _HID), jnp.bfloat16)] * 2,
    )(hsums, h)

    agg1p, cnt1p = sc_agg1(hb, src1, dst1, zr1)

    out = pl.pallas_call(
        _final_body,
        grid=(_ND1 // _FBLK,),
        in_specs=[pl.BlockSpec((_FBLK, _HID), lambda i: (i, 0)),
                  pl.BlockSpec((_FBLK, _HID), lambda i: (i, 0)),
                  pl.BlockSpec((_FBLK, 1), lambda i: (i, 0)),
                  pl.BlockSpec((_FBLK, 1), lambda i: (i, 0)),
                  pl.BlockSpec((_FBLK, _HID), lambda i: (i, 0)),
                  pl.BlockSpec((_HID, _OUT), lambda i: (0, 0)),
                  pl.BlockSpec((_HID, _OUT), lambda i: (0, 0)),
                  pl.BlockSpec((1, _OUT), lambda i: (0, 0))],
        out_specs=pl.BlockSpec((_FBLK, _OUT), lambda i: (i, 0)),
        out_shape=jax.ShapeDtypeStruct((_ND1, _OUT), f32),
    )(agg1p[:_ND1], agg1p[_ND1:], cnt1p[:_ND1].reshape(_ND1, 1),
      cnt1p[_ND1:].reshape(_ND1, 1), htb, W_l1, W_r1, b1.reshape(1, _OUT))
    return out


# speedup vs baseline: 2.3286x; 1.0109x over previous
import functools

import jax
import jax.numpy as jnp
from jax import lax
from jax.experimental import pallas as pl
from jax.experimental.pallas import tpu as pltpu
from jax.experimental.pallas import tpu_sc as plsc

_IN_CH = 128
_HID = 128
_OUT = 64
_N_SRC = 100000
_ND0 = 20000
_ND1 = 4096
_E0 = 320000
_E1 = 65536
_NC = 2
_NS = 16

_CH = 128
_NRING = 8

_HALF0 = _ND0 // 2
_ROWS0 = 10240
_GARB0 = _HALF0
_EPT0 = 20480
_E0P = _EPT0 * _NS
_NCH0 = _EPT0 // _CH

_EPC1 = _E1 // _NC
_EPT1 = _EPC1 // _NS
_NCH1 = _EPT1 // _CH

_CBLK = 2000
_BBLK = 2000
_HBLK = 400
_FBLK = 512


def _colsum_body(x_ref, out_ref):
    i = pl.program_id(0)

    @pl.when(i == 0)
    def _():
        out_ref[...] = jnp.zeros_like(out_ref)

    ssum = jnp.sum(x_ref[...], axis=0, keepdims=True)
    out_ref[0:1, :] += ssum

    @pl.when(i < _ND0 // _CBLK)
    def _():
        out_ref[1:2, :] += ssum


def _binarize_body(sums_ref, x_ref, xb_ref, xtb_ref, *, nfull, npart):
    m_full = sums_ref[0:1, :] / nfull
    m_part = sums_ref[1:2, :] / npart
    blk = x_ref[...]
    xb_ref[...] = jnp.sign(blk - m_full).astype(xb_ref.dtype)
    xtb_ref[...] = jnp.sign(blk - m_part).astype(xtb_ref.dtype)


_NH1 = _ND0 // _HBLK
_NH2 = _ND1 // _FBLK
_HKEEP = 4400


def _layer0_body(aggA_ref, aggB_ref, cnt_ref, xtb_ref, wlA_ref, wlB_ref,
                 wr_ref, b_ref, hb_ref, htb_ref, h_sc, hsums_sc):
    i = pl.program_id(0)

    @pl.when(i == 0)
    def _():
        hsums_sc[...] = jnp.zeros_like(hsums_sc)

    @pl.when(i < _NH1)
    def _():
        cnt = jnp.maximum(cnt_ref[...], 1.0)
        aggA = aggA_ref[...].astype(jnp.float32)
        aggB = aggB_ref[...].astype(jnp.float32)
        hblk = (jnp.dot(aggA / cnt, wlA_ref[...],
                        preferred_element_type=jnp.float32)
                + jnp.dot(aggB / cnt, wlB_ref[...],
                          preferred_element_type=jnp.float32)
                + jnp.dot(xtb_ref[...].astype(jnp.float32), wr_ref[...],
                          preferred_element_type=jnp.float32)
                + b_ref[...])
        hblk = jnp.maximum(hblk, 0.0)

        @pl.when(i < _HKEEP // _HBLK)
        def _():
            h_sc[pl.ds(i * _HBLK, _HBLK), :] = hblk

        hsums_sc[0:1, :] += jnp.sum(hblk, axis=0, keepdims=True)
        rows = i * _HBLK + lax.broadcasted_iota(jnp.int32, (_HBLK, 1), 0)
        hsums_sc[1:2, :] += jnp.sum(jnp.where(rows < _ND1, hblk, 0.0),
                                    axis=0, keepdims=True)

    @pl.when(i >= _NH1)
    def _():
        j = i - _NH1
        m_full = hsums_sc[0:1, :] / float(_ND0)
        m_part = hsums_sc[1:2, :] / float(_ND1)
        blk = h_sc[pl.ds(j * _FBLK, _FBLK), :]
        hb_ref[...] = jnp.sign(blk - m_full).astype(hb_ref.dtype)
        htb_ref[...] = jnp.sign(blk - m_part).astype(htb_ref.dtype)


def _final_body(aggA_ref, aggB_ref, cntA_ref, cntB_ref, htb_ref,
                wl_ref, wr_ref, b_ref, out_ref):
    cnt = jnp.maximum(cntA_ref[...] + cntB_ref[...], 1.0)
    ma = (aggA_ref[...].astype(jnp.float32)
          + aggB_ref[...].astype(jnp.float32)) / cnt
    z = (jnp.dot(ma, wl_ref[...], preferred_element_type=jnp.float32)
         + jnp.dot(htb_ref[...].astype(jnp.float32), wr_ref[...],
                   preferred_element_type=jnp.float32)
         + b_ref[...])
    z = z - jnp.max(z, axis=1, keepdims=True)
    z = z - jnp.log(jnp.sum(jnp.exp(z), axis=1, keepdims=True))
    out_ref[...] = z


_IB = 1024
_CPB = _IB // _CH
_CAP0 = _EPT0 + 2 * _CH


def _make_agg_body(ept, nch, half, garb, z_stripe, out_rows, out_tiles,
                   core_splits_edges, col_split=False):
    nb = ept // _IB
    assert nb % 2 == 0 or nb == 1
    assert nch == nb * _CPB
    assert _CPB >= _NRING
    cw = (max(z_stripe, out_rows) + 15) // 16 * 16

    def body(tab, src, dst, zr, agg_out, cnt_out, *sc):
        srcb = [sc[0], sc[1]]
        dstb = [sc[2], sc[3]]
        rows = list(sc[4:4 + _NRING])
        sidx = list(sc[4 + _NRING:4 + 2 * _NRING])
        gidx = list(sc[4 + 2 * _NRING:4 + 3 * _NRING])
        ones_v, cstage_v, agg_sh, cnt_sh = sc[4 + 3 * _NRING:8 + 3 * _NRING]
        gsem = list(sc[8 + 3 * _NRING:8 + 4 * _NRING])
        isem = list(sc[8 + 4 * _NRING:10 + 4 * _NRING])
        c = lax.axis_index("c")
        s = lax.axis_index("s")
        dbase = 0 if (core_splits_edges or col_split) else c * half
        ebase = (c * (ept * _NS) if core_splits_edges else 0) + s * ept

        def zbody(i, carry):
            cstage_v[pl.ds(i * 16, 16)] = jnp.zeros((16,), jnp.float32)
            return carry

        lax.fori_loop(0, cw // 16, zbody, 0)
        pltpu.sync_copy(zr.at[pl.ds(s * z_stripe, z_stripe)],
                        agg_sh.at[pl.ds(s * z_stripe, z_stripe)])
        pltpu.sync_copy(cstage_v.at[pl.ds(0, z_stripe)],
                        cnt_sh.at[pl.ds(s * z_stripe, z_stripe)])
        for k in range(_CH // 16):
            ones_v[pl.ds(k * 16, 16)] = jnp.full((16,), 1.0, jnp.float32)
        pltpu.sync_copy(src.at[pl.ds(ebase, _IB)], srcb[0])
        pltpu.sync_copy(dst.at[pl.ds(ebase, _IB)], dstb[0])
        if nb > 1:
            pltpu.async_copy(src.at[pl.ds(ebase + _IB, _IB)], srcb[1], isem[1])
            pltpu.async_copy(dst.at[pl.ds(ebase + _IB, _IB)], dstb[1], isem[1])
        plsc.subcore_barrier()

        garb_u = jnp.uint32(garb)

        def compute_sidx(dbuf, off, xbuf):
            for k in range(_CH // 16):
                d = dbuf[pl.ds(off + k * 16, 16)]
                loc = plsc.bitcast(d - dbase, jnp.uint32)
                xbuf[pl.ds(k * 16, 16)] = plsc.bitcast(
                    jnp.minimum(loc, garb_u), jnp.int32)

        def gather(sbuf, off, b):
            if not col_split:
                return pltpu.async_copy(
                    tab.at[sbuf.at[pl.ds(off, _CH)]], rows[b], gsem[b])
            for k in range(_CH // 16):
                sv = sbuf[pl.ds(off + k * 16, 16)]
                gidx[b][pl.ds(k * 16, 16)] = (sv << 1) + c
            return pltpu.async_copy(tab.at[gidx[b]], rows[b], gsem[b])

        def gather_wait(sbuf, off, b):
            if not col_split:
                pltpu.make_async_copy(
                    tab.at[sbuf.at[pl.ds(off, _CH)]], rows[b], gsem[b]).wait()
            else:
                pltpu.make_async_copy(tab.at[gidx[b]], rows[b], gsem[b]).wait()

        def scatter(b):
            pltpu.sync_copy(rows[b], agg_sh.at[sidx[b]], add=True)
            if col_split:
                @pl.when(c == 0)
                def _():
                    pltpu.sync_copy(ones_v, cnt_sh.at[sidx[b]], add=True)
            else:
                pltpu.sync_copy(ones_v, cnt_sh.at[sidx[b]], add=True)

        for b in range(_NRING):
            compute_sidx(dstb[0], b * _CH, sidx[b])
            gather(srcb[0], b * _CH, b)

        def pair_body(t, carry):
            for hm in range(2):
                m = 2 * t + hm
                bufm = hm
                bufn = 1 - hm
                for p in range(_CPB):
                    b = p % _NRING
                    gather_wait(srcb[bufm], p * _CH, b)
                    scatter(b)
                    if p == _CPB - _NRING:
                        @pl.when(m + 2 < nb)
                        def _():
                            off = ebase + (m + 2) * _IB
                            pltpu.async_copy(src.at[pl.ds(off, _IB)],
                                             srcb[bufm], isem[bufm])
                            pltpu.async_copy(dst.at[pl.ds(off, _IB)],
                                             dstb[bufm], isem[bufm])

                        @pl.when(m + 1 < nb)
                        def _():
                            off = ebase + (m + 1) * _IB
                            pltpu.make_async_copy(
                                src.at[pl.ds(off, _IB)], srcb[bufn],
                                isem[bufn]).wait()
                            pltpu.make_async_copy(
                                dst.at[pl.ds(off, _IB)], dstb[bufn],
                                isem[bufn]).wait()
                    j = m * _CPB + p + _NRING
                    if p < _CPB - _NRING:
                        jbuf, joff = bufm, (p + _NRING) * _CH
                    else:
                        jbuf, joff = bufn, (p + _NRING - _CPB) * _CH

                    @pl.when(j < nch)
                    def _():
                        compute_sidx(dstb[jbuf], joff, sidx[b])
                        gather(srcb[jbuf], joff, b)
            return carry

        lax.fori_loop(0, max(nb // 2, 1), pair_body, 0)
        plsc.subcore_barrier()

        @pl.when(s < out_tiles)
        def _():
            pltpu.sync_copy(agg_sh.at[pl.ds(s * out_rows, out_rows)],
                            agg_out.at[pl.ds(c * half + s * out_rows, out_rows)])

        cnt_base = (s * out_rows) if col_split else (c * half + s * out_rows)
        cnt_write = ((c == 0) & (s < out_tiles)) if col_split else (s < out_tiles)

        @pl.when(cnt_write)
        def _():
            pltpu.sync_copy(cnt_sh.at[pl.ds(s * out_rows, out_rows)],
                            cstage_v.at[pl.ds(0, out_rows)])
            pltpu.sync_copy(cstage_v.at[pl.ds(0, out_rows)],
                            cnt_out.at[pl.ds(cnt_base, out_rows)])

    return body


def _agg_scratch(rows, cstage, w, dtype=jnp.float32):
    return ([pltpu.VMEM((_IB,), jnp.int32)] * 2
            + [pltpu.VMEM((_IB,), jnp.int32)] * 2
            + [pltpu.VMEM((_CH, w), dtype)] * _NRING
            + [pltpu.VMEM((_CH,), jnp.int32)] * _NRING
            + [pltpu.VMEM((_CH,), jnp.int32)] * _NRING
            + [pltpu.VMEM((_CH,), jnp.float32),
               pltpu.VMEM((cstage,), jnp.float32),
               pltpu.VMEM_SHARED((rows, w), dtype),
               pltpu.VMEM_SHARED((rows,), jnp.float32)]
            + [pltpu.SemaphoreType.DMA] * (_NRING + 2))


_ROWS0C = 20480


@functools.cache
def _sc_kernels():
    mesh = plsc.VectorSubcoreMesh(core_axis_name="c", subcore_axis_name="s",
                                  num_cores=_NC, num_subcores=_NS)
    agg0 = pl.kernel(
        _make_agg_body(_EPT0, _NCH0, _ND0, _ROWS0C - 1, z_stripe=1280,
                       out_rows=2000, out_tiles=10, core_splits_edges=False,
                       col_split=True),
        out_type=[jax.ShapeDtypeStruct((_NC * _ND0, _IN_CH // 2), jnp.bfloat16),
                  jax.ShapeDtypeStruct((_ND0,), jnp.float32)],
        mesh=mesh,
        scratch_types=_agg_scratch(_ROWS0C, 2000, _IN_CH // 2, jnp.bfloat16),
        compiler_params=pltpu.CompilerParams(use_tc_tiling_on_sc=False),
    )
    agg1 = pl.kernel(
        _make_agg_body(_EPT1, _NCH1, _ND1, _ND1 - 1, z_stripe=256,
                       out_rows=256, out_tiles=_NS, core_splits_edges=True),
        out_type=[jax.ShapeDtypeStruct((_NC * _ND1, _HID), jnp.bfloat16),
                  jax.ShapeDtypeStruct((_NC * _ND1,), jnp.float32)],
        mesh=mesh,
        scratch_types=_agg_scratch(_ND1, 256, _HID, jnp.bfloat16),
        compiler_params=pltpu.CompilerParams(use_tc_tiling_on_sc=False),
    )
    return agg0, agg1


def kernel(x, edge_index0, edge_index1, size0_dst, size1_dst,
           W_l0, W_r0, b0, W_l1, W_r1, b1):
    f32 = jnp.float32
    x = x.astype(f32)
    src0 = edge_index0[0]
    dst0 = edge_index0[1]
    src1 = edge_index1[0]
    dst1 = edge_index1[1]

    sums = pl.pallas_call(
        _colsum_body,
        grid=(_N_SRC // _CBLK,),
        in_specs=[pl.BlockSpec((_CBLK, _IN_CH), lambda i: (i, 0))],
        out_specs=pl.BlockSpec((8, _IN_CH), lambda i: (0, 0)),
        out_shape=jax.ShapeDtypeStruct((8, _IN_CH), f32),
    )(x)

    xb, xtb = pl.pallas_call(
        functools.partial(_binarize_body, nfull=float(_N_SRC), npart=float(_ND0)),
        grid=(_ND0 // _BBLK,),
        in_specs=[pl.BlockSpec((8, _IN_CH), lambda i: (0, 0)),
                  pl.BlockSpec((_BBLK, _IN_CH), lambda i: (i, 0))],
        out_specs=[pl.BlockSpec((_BBLK, _IN_CH), lambda i: (i, 0))] * 2,
        out_shape=[jax.ShapeDtypeStruct((_ND0, _IN_CH), jnp.bfloat16)] * 2,
    )(sums, x)

    zr0 = jnp.zeros((_ROWS0C, _IN_CH // 2), jnp.bfloat16)
    zr1 = jnp.zeros((_NS * 256, _HID), jnp.bfloat16)
    npad = _E0P - _E0
    src0p = jnp.concatenate([src0, jnp.zeros((npad,), jnp.int32)])
    dst0p = jnp.concatenate([dst0, jnp.full((npad,), _ND0, jnp.int32)])
    sc_agg0, sc_agg1 = _sc_kernels()
    agg0p, cnt0 = sc_agg0(xb.reshape(_NC * _ND0, _IN_CH // 2), src0p, dst0p,
                          zr0)

    p1 = lambda i: (jnp.minimum(i, _NH1 - 1), 0)
    p2 = lambda i: (jnp.maximum(i - _NH1, 0), 0)
    hb, htb = pl.pallas_call(
        _layer0_body,
        grid=(_NH1 + _NH2,),
        in_specs=[pl.BlockSpec((_HBLK, _IN_CH // 2), p1),
                  pl.BlockSpec((_HBLK, _IN_CH // 2), p1),
                  pl.BlockSpec((_HBLK, 1), p1),
                  pl.BlockSpec((_HBLK, _IN_CH), p1),
                  pl.BlockSpec((_IN_CH // 2, _HID), lambda i: (0, 0)),
                  pl.BlockSpec((_IN_CH // 2, _HID), lambda i: (0, 0)),
                  pl.BlockSpec((_IN_CH, _HID), lambda i: (0, 0)),
                  pl.BlockSpec((1, _HID), lambda i: (0, 0))],
        out_specs=[pl.BlockSpec((_FBLK, _HID), p2)] * 2,
        out_shape=[jax.ShapeDtypeStruct((_ND1, _HID), jnp.bfloat16)] * 2,
        scratch_shapes=[pltpu.VMEM((_HKEEP, _HID), f32),
                        pltpu.VMEM((8, _HID), f32)],
    )(agg0p[:_ND0], agg0p[_ND0:], cnt0.reshape(_ND0, 1), xtb,
      W_l0[:_IN_CH // 2], W_l0[_IN_CH // 2:], W_r0, b0.reshape(1, _HID))

    agg1p, cnt1p = sc_agg1(hb, src1, dst1, zr1)

    out = pl.pallas_call(
        _final_body,
        grid=(_ND1 // _FBLK,),
        in_specs=[pl.BlockSpec((_FBLK, _HID), lambda i: (i, 0)),
                  pl.BlockSpec((_FBLK, _HID), lambda i: (i, 0)),
                  pl.BlockSpec((_FBLK, 1), lambda i: (i, 0)),
                  pl.BlockSpec((_FBLK, 1), lambda i: (i, 0)),
                  pl.BlockSpec((_FBLK, _HID), lambda i: (i, 0)),
                  pl.BlockSpec((_HID, _OUT), lambda i: (0, 0)),
                  pl.BlockSpec((_HID, _OUT), lambda i: (0, 0)),
                  pl.BlockSpec((1, _OUT), lambda i: (0, 0))],
        out_specs=pl.BlockSpec((_FBLK, _OUT), lambda i: (i, 0)),
        out_shape=jax.ShapeDtypeStruct((_ND1, _OUT), f32),
    )(agg1p[:_ND1], agg1p[_ND1:], cnt1p[:_ND1].reshape(_ND1, 1),
      cnt1p[_ND1:].reshape(_ND1, 1), htb, W_l1, W_r1, b1.reshape(1, _OUT))
    return out


# speedup vs baseline: 2.3290x; 1.0001x over previous
import functools

import jax
import jax.numpy as jnp
from jax import lax
from jax.experimental import pallas as pl
from jax.experimental.pallas import tpu as pltpu
from jax.experimental.pallas import tpu_sc as plsc

_IN_CH = 128
_HID = 128
_OUT = 64
_N_SRC = 100000
_ND0 = 20000
_ND1 = 4096
_E0 = 320000
_E1 = 65536
_NC = 2
_NS = 16

_CH = 128
_NRING = 8

_HALF0 = _ND0 // 2
_ROWS0 = 10240
_GARB0 = _HALF0
_EPT0 = 20480
_E0P = _EPT0 * _NS
_NCH0 = _EPT0 // _CH

_EPC1 = _E1 // _NC
_EPT1 = _EPC1 // _NS
_NCH1 = _EPT1 // _CH

_CBLK = 2000
_BBLK = 2000
_HBLK = 400
_FBLK = 512


_NC1 = _N_SRC // _CBLK
_NC2 = _ND0 // _BBLK


def _colsum_binarize_body(x_ref, xb_ref, xtb_ref, sums_sc):
    i = pl.program_id(0)

    @pl.when(i == 0)
    def _():
        sums_sc[...] = jnp.zeros_like(sums_sc)

    @pl.when(i < _NC1)
    def _():
        ssum = jnp.sum(x_ref[...], axis=0, keepdims=True)
        sums_sc[0:1, :] += ssum

        @pl.when(i < _ND0 // _CBLK)
        def _():
            sums_sc[1:2, :] += ssum

    @pl.when(i >= _NC1)
    def _():
        m_full = sums_sc[0:1, :] / float(_N_SRC)
        m_part = sums_sc[1:2, :] / float(_ND0)
        blk = x_ref[...]
        xb_ref[...] = jnp.sign(blk - m_full).astype(xb_ref.dtype)
        xtb_ref[...] = jnp.sign(blk - m_part).astype(xtb_ref.dtype)


def _binarize_body(sums_ref, x_ref, xb_ref, xtb_ref, *, nfull, npart):
    m_full = sums_ref[0:1, :] / nfull
    m_part = sums_ref[1:2, :] / npart
    blk = x_ref[...]
    xb_ref[...] = jnp.sign(blk - m_full).astype(xb_ref.dtype)
    xtb_ref[...] = jnp.sign(blk - m_part).astype(xtb_ref.dtype)


_NH1 = _ND0 // _HBLK
_NH2 = _ND1 // _FBLK
_HKEEP = 4400


def _layer0_body(aggA_ref, aggB_ref, cnt_ref, xtb_ref, wlA_ref, wlB_ref,
                 wr_ref, b_ref, hb_ref, htb_ref, h_sc, hsums_sc):
    i = pl.program_id(0)

    @pl.when(i == 0)
    def _():
        hsums_sc[...] = jnp.zeros_like(hsums_sc)

    @pl.when(i < _NH1)
    def _():
        cnt = jnp.maximum(cnt_ref[...], 1.0)
        aggA = aggA_ref[...].astype(jnp.float32)
        aggB = aggB_ref[...].astype(jnp.float32)
        hblk = (jnp.dot(aggA / cnt, wlA_ref[...],
                        preferred_element_type=jnp.float32)
                + jnp.dot(aggB / cnt, wlB_ref[...],
                          preferred_element_type=jnp.float32)
                + jnp.dot(xtb_ref[...].astype(jnp.float32), wr_ref[...],
                          preferred_element_type=jnp.float32)
                + b_ref[...])
        hblk = jnp.maximum(hblk, 0.0)

        @pl.when(i < _HKEEP // _HBLK)
        def _():
            h_sc[pl.ds(i * _HBLK, _HBLK), :] = hblk

        hsums_sc[0:1, :] += jnp.sum(hblk, axis=0, keepdims=True)
        rows = i * _HBLK + lax.broadcasted_iota(jnp.int32, (_HBLK, 1), 0)
        hsums_sc[1:2, :] += jnp.sum(jnp.where(rows < _ND1, hblk, 0.0),
                                    axis=0, keepdims=True)

    @pl.when(i >= _NH1)
    def _():
        j = i - _NH1
        m_full = hsums_sc[0:1, :] / float(_ND0)
        m_part = hsums_sc[1:2, :] / float(_ND1)
        blk = h_sc[pl.ds(j * _FBLK, _FBLK), :]
        hb_ref[...] = jnp.sign(blk - m_full).astype(hb_ref.dtype)
        htb_ref[...] = jnp.sign(blk - m_part).astype(htb_ref.dtype)


def _final_body(aggA_ref, aggB_ref, cntA_ref, cntB_ref, htb_ref,
                wl_ref, wr_ref, b_ref, out_ref):
    cnt = jnp.maximum(cntA_ref[...] + cntB_ref[...], 1.0)
    ma = (aggA_ref[...].astype(jnp.float32)
          + aggB_ref[...].astype(jnp.float32)) / cnt
    z = (jnp.dot(ma, wl_ref[...], preferred_element_type=jnp.float32)
         + jnp.dot(htb_ref[...].astype(jnp.float32), wr_ref[...],
                   preferred_element_type=jnp.float32)
         + b_ref[...])
    z = z - jnp.max(z, axis=1, keepdims=True)
    z = z - jnp.log(jnp.sum(jnp.exp(z), axis=1, keepdims=True))
    out_ref[...] = z


_IB = 1024
_CPB = _IB // _CH
_CAP0 = _EPT0 + 2 * _CH


def _make_agg_body(ept, nch, half, garb, z_stripe, out_rows, out_tiles,
                   core_splits_edges, col_split=False):
    nb = ept // _IB
    assert nb % 2 == 0 or nb == 1
    assert nch == nb * _CPB
    assert _CPB >= _NRING
    cw = (max(z_stripe, out_rows) + 15) // 16 * 16

    def body(tab, src, dst, zr, agg_out, cnt_out, *sc):
        srcb = [sc[0], sc[1]]
        dstb = [sc[2], sc[3]]
        rows = list(sc[4:4 + _NRING])
        sidx = list(sc[4 + _NRING:4 + 2 * _NRING])
        gidx = list(sc[4 + 2 * _NRING:4 + 3 * _NRING])
        ones_v, cstage_v, agg_sh, cnt_sh = sc[4 + 3 * _NRING:8 + 3 * _NRING]
        gsem = list(sc[8 + 3 * _NRING:8 + 4 * _NRING])
        isem = list(sc[8 + 4 * _NRING:10 + 4 * _NRING])
        c = lax.axis_index("c")
        s = lax.axis_index("s")
        dbase = 0 if (core_splits_edges or col_split) else c * half
        ebase = (c * (ept * _NS) if core_splits_edges else 0) + s * ept

        def zbody(i, carry):
            cstage_v[pl.ds(i * 16, 16)] = jnp.zeros((16,), jnp.float32)
            return carry

        lax.fori_loop(0, cw // 16, zbody, 0)
        pltpu.sync_copy(zr.at[pl.ds(s * z_stripe, z_stripe)],
                        agg_sh.at[pl.ds(s * z_stripe, z_stripe)])
        pltpu.sync_copy(cstage_v.at[pl.ds(0, z_stripe)],
                        cnt_sh.at[pl.ds(s * z_stripe, z_stripe)])
        for k in range(_CH // 16):
            ones_v[pl.ds(k * 16, 16)] = jnp.full((16,), 1.0, jnp.float32)
        pltpu.sync_copy(src.at[pl.ds(ebase, _IB)], srcb[0])
        pltpu.sync_copy(dst.at[pl.ds(ebase, _IB)], dstb[0])
        if nb > 1:
            pltpu.async_copy(src.at[pl.ds(ebase + _IB, _IB)], srcb[1], isem[1])
            pltpu.async_copy(dst.at[pl.ds(ebase + _IB, _IB)], dstb[1], isem[1])
        plsc.subcore_barrier()

        garb_u = jnp.uint32(garb)

        def compute_sidx(dbuf, off, xbuf):
            for k in range(_CH // 16):
                d = dbuf[pl.ds(off + k * 16, 16)]
                loc = plsc.bitcast(d - dbase, jnp.uint32)
                xbuf[pl.ds(k * 16, 16)] = plsc.bitcast(
                    jnp.minimum(loc, garb_u), jnp.int32)

        def gather(sbuf, off, b):
            if not col_split:
                return pltpu.async_copy(
                    tab.at[sbuf.at[pl.ds(off, _CH)]], rows[b], gsem[b])
            for k in range(_CH // 16):
                sv = sbuf[pl.ds(off + k * 16, 16)]
                gidx[b][pl.ds(k * 16, 16)] = (sv << 1) + c
            return pltpu.async_copy(tab.at[gidx[b]], rows[b], gsem[b])

        def gather_wait(sbuf, off, b):
            if not col_split:
                pltpu.make_async_copy(
                    tab.at[sbuf.at[pl.ds(off, _CH)]], rows[b], gsem[b]).wait()
            else:
                pltpu.make_async_copy(tab.at[gidx[b]], rows[b], gsem[b]).wait()

        def scatter(b):
            pltpu.sync_copy(rows[b], agg_sh.at[sidx[b]], add=True)
            if col_split:
                @pl.when(c == 0)
                def _():
                    pltpu.sync_copy(ones_v, cnt_sh.at[sidx[b]], add=True)
            else:
                pltpu.sync_copy(ones_v, cnt_sh.at[sidx[b]], add=True)

        for b in range(_NRING):
            compute_sidx(dstb[0], b * _CH, sidx[b])
            gather(srcb[0], b * _CH, b)

        def pair_body(t, carry):
            for hm in range(2):
                m = 2 * t + hm
                bufm = hm
                bufn = 1 - hm
                for p in range(_CPB):
                    b = p % _NRING
                    gather_wait(srcb[bufm], p * _CH, b)
                    scatter(b)
                    if p == _CPB - _NRING:
                        @pl.when(m + 2 < nb)
                        def _():
                            off = ebase + (m + 2) * _IB
                            pltpu.async_copy(src.at[pl.ds(off, _IB)],
                                             srcb[bufm], isem[bufm])
                            pltpu.async_copy(dst.at[pl.ds(off, _IB)],
                                             dstb[bufm], isem[bufm])

                        @pl.when(m + 1 < nb)
                        def _():
                            off = ebase + (m + 1) * _IB
                            pltpu.make_async_copy(
                                src.at[pl.ds(off, _IB)], srcb[bufn],
                                isem[bufn]).wait()
                            pltpu.make_async_copy(
                                dst.at[pl.ds(off, _IB)], dstb[bufn],
                                isem[bufn]).wait()
                    j = m * _CPB + p + _NRING
                    if p < _CPB - _NRING:
                        jbuf, joff = bufm, (p + _NRING) * _CH
                    else:
                        jbuf, joff = bufn, (p + _NRING - _CPB) * _CH

                    @pl.when(j < nch)
                    def _():
                        compute_sidx(dstb[jbuf], joff, sidx[b])
                        gather(srcb[jbuf], joff, b)
            return carry

        lax.fori_loop(0, max(nb // 2, 1), pair_body, 0)
        plsc.subcore_barrier()

        @pl.when(s < out_tiles)
        def _():
            pltpu.sync_copy(agg_sh.at[pl.ds(s * out_rows, out_rows)],
                            agg_out.at[pl.ds(c * half + s * out_rows, out_rows)])

        cnt_base = (s * out_rows) if col_split else (c * half + s * out_rows)
        cnt_write = ((c == 0) & (s < out_tiles)) if col_split else (s < out_tiles)

        @pl.when(cnt_write)
        def _():
            pltpu.sync_copy(cnt_sh.at[pl.ds(s * out_rows, out_rows)],
                            cstage_v.at[pl.ds(0, out_rows)])
            pltpu.sync_copy(cstage_v.at[pl.ds(0, out_rows)],
                            cnt_out.at[pl.ds(cnt_base, out_rows)])

    return body


def _agg_scratch(rows, cstage, w, dtype=jnp.float32):
    return ([pltpu.VMEM((_IB,), jnp.int32)] * 2
            + [pltpu.VMEM((_IB,), jnp.int32)] * 2
            + [pltpu.VMEM((_CH, w), dtype)] * _NRING
            + [pltpu.VMEM((_CH,), jnp.int32)] * _NRING
            + [pltpu.VMEM((_CH,), jnp.int32)] * _NRING
            + [pltpu.VMEM((_CH,), jnp.float32),
               pltpu.VMEM((cstage,), jnp.float32),
               pltpu.VMEM_SHARED((rows, w), dtype),
               pltpu.VMEM_SHARED((rows,), jnp.float32)]
            + [pltpu.SemaphoreType.DMA] * (_NRING + 2))


_ROWS0C = 20480


@functools.cache
def _sc_kernels():
    mesh = plsc.VectorSubcoreMesh(core_axis_name="c", subcore_axis_name="s",
                                  num_cores=_NC, num_subcores=_NS)
    agg0 = pl.kernel(
        _make_agg_body(_EPT0, _NCH0, _ND0, _ROWS0C - 1, z_stripe=1280,
                       out_rows=2000, out_tiles=10, core_splits_edges=False,
                       col_split=True),
        out_type=[jax.ShapeDtypeStruct((_NC * _ND0, _IN_CH // 2), jnp.bfloat16),
                  jax.ShapeDtypeStruct((_ND0,), jnp.float32)],
        mesh=mesh,
        scratch_types=_agg_scratch(_ROWS0C, 2000, _IN_CH // 2, jnp.bfloat16),
        compiler_params=pltpu.CompilerParams(use_tc_tiling_on_sc=False),
    )
    agg1 = pl.kernel(
        _make_agg_body(_EPT1, _NCH1, _ND1, _ND1 - 1, z_stripe=256,
                       out_rows=256, out_tiles=_NS, core_splits_edges=True),
        out_type=[jax.ShapeDtypeStruct((_NC * _ND1, _HID), jnp.bfloat16),
                  jax.ShapeDtypeStruct((_NC * _ND1,), jnp.float32)],
        mesh=mesh,
        scratch_types=_agg_scratch(_ND1, 256, _HID, jnp.bfloat16),
        compiler_params=pltpu.CompilerParams(use_tc_tiling_on_sc=False),
    )
    return agg0, agg1


def kernel(x, edge_index0, edge_index1, size0_dst, size1_dst,
           W_l0, W_r0, b0, W_l1, W_r1, b1):
    f32 = jnp.float32
    x = x.astype(f32)
    src0 = edge_index0[0]
    dst0 = edge_index0[1]
    src1 = edge_index1[0]
    dst1 = edge_index1[1]

    xb, xtb = pl.pallas_call(
        _colsum_binarize_body,
        grid=(_NC1 + _NC2,),
        in_specs=[pl.BlockSpec(
            (_CBLK, _IN_CH),
            lambda i: (jnp.where(i < _NC1, i, i - _NC1), 0))],
        out_specs=[pl.BlockSpec(
            (_BBLK, _IN_CH),
            lambda i: (jnp.maximum(i - _NC1, 0), 0))] * 2,
        out_shape=[jax.ShapeDtypeStruct((_ND0, _IN_CH), jnp.bfloat16)] * 2,
        scratch_shapes=[pltpu.VMEM((8, _IN_CH), f32)],
    )(x)

    zr0 = jnp.zeros((_ROWS0C, _IN_CH // 2), jnp.bfloat16)
    zr1 = jnp.zeros((_NS * 256, _HID), jnp.bfloat16)
    npad = _E0P - _E0
    src0p = jnp.concatenate([src0, jnp.zeros((npad,), jnp.int32)])
    dst0p = jnp.concatenate([dst0, jnp.full((npad,), _ND0, jnp.int32)])
    sc_agg0, sc_agg1 = _sc_kernels()
    agg0p, cnt0 = sc_agg0(xb.reshape(_NC * _ND0, _IN_CH // 2), src0p, dst0p,
                          zr0)

    p1 = lambda i: (jnp.minimum(i, _NH1 - 1), 0)
    p2 = lambda i: (jnp.maximum(i - _NH1, 0), 0)
    hb, htb = pl.pallas_call(
        _layer0_body,
        grid=(_NH1 + _NH2,),
        in_specs=[pl.BlockSpec((_HBLK, _IN_CH // 2), p1),
                  pl.BlockSpec((_HBLK, _IN_CH // 2), p1),
                  pl.BlockSpec((_HBLK, 1), p1),
                  pl.BlockSpec((_HBLK, _IN_CH), p1),
                  pl.BlockSpec((_IN_CH // 2, _HID), lambda i: (0, 0)),
                  pl.BlockSpec((_IN_CH // 2, _HID), lambda i: (0, 0)),
                  pl.BlockSpec((_IN_CH, _HID), lambda i: (0, 0)),
                  pl.BlockSpec((1, _HID), lambda i: (0, 0))],
        out_specs=[pl.BlockSpec((_FBLK, _HID), p2)] * 2,
        out_shape=[jax.ShapeDtypeStruct((_ND1, _HID), jnp.bfloat16)] * 2,
        scratch_shapes=[pltpu.VMEM((_HKEEP, _HID), f32),
                        pltpu.VMEM((8, _HID), f32)],
    )(agg0p[:_ND0], agg0p[_ND0:], cnt0.reshape(_ND0, 1), xtb,
      W_l0[:_IN_CH // 2], W_l0[_IN_CH // 2:], W_r0, b0.reshape(1, _HID))

    agg1p, cnt1p = sc_agg1(hb, src1, dst1, zr1)

    out = pl.pallas_call(
        _final_body,
        grid=(_ND1 // _FBLK,),
        in_specs=[pl.BlockSpec((_FBLK, _HID), lambda i: (i, 0)),
                  pl.BlockSpec((_FBLK, _HID), lambda i: (i, 0)),
                  pl.BlockSpec((_FBLK, 1), lambda i: (i, 0)),
                  pl.BlockSpec((_FBLK, 1), lambda i: (i, 0)),
                  pl.BlockSpec((_FBLK, _HID), lambda i: (i, 0)),
                  pl.BlockSpec((_HID, _OUT), lambda i: (0, 0)),
                  pl.BlockSpec((_HID, _OUT), lambda i: (0, 0)),
                  pl.BlockSpec((1, _OUT), lambda i: (0, 0))],
        out_specs=pl.BlockSpec((_FBLK, _OUT), lambda i: (i, 0)),
        out_shape=jax.ShapeDtypeStruct((_ND1, _OUT), f32),
    )(agg1p[:_ND1], agg1p[_ND1:], cnt1p[:_ND1].reshape(_ND1, 1),
      cnt1p[_ND1:].reshape(_ND1, 1), htb, W_l1, W_r1, b1.reshape(1, _OUT))
    return out


# speedup vs baseline: 2.3303x; 1.0006x over previous
import functools

import jax
import jax.numpy as jnp
from jax import lax
from jax.experimental import pallas as pl
from jax.experimental.pallas import tpu as pltpu
from jax.experimental.pallas import tpu_sc as plsc

_IN_CH = 128
_HID = 128
_OUT = 64
_N_SRC = 100000
_ND0 = 20000
_ND1 = 4096
_E0 = 320000
_E1 = 65536
_NC = 2
_NS = 16

_CH = 128
_NRING = 8

_HALF0 = _ND0 // 2
_ROWS0 = 10240
_GARB0 = _HALF0
_EPT0 = 20480
_E0P = _EPT0 * _NS
_NCH0 = _EPT0 // _CH

_EPC1 = _E1 // _NC
_EPT1 = _EPC1 // _NS
_NCH1 = _EPT1 // _CH

_CBLK = 2000
_BBLK = 2000
_HBLK = 400
_FBLK = 512


def _colsum_body(x_ref, out_ref):
    i = pl.program_id(0)

    @pl.when(i == 0)
    def _():
        out_ref[...] = jnp.zeros_like(out_ref)

    ssum = jnp.sum(x_ref[...], axis=0, keepdims=True)
    out_ref[0:1, :] += ssum

    @pl.when(i < _ND0 // _CBLK)
    def _():
        out_ref[1:2, :] += ssum


def _binarize_body(sums_ref, x_ref, xb_ref, xtb_ref, *, nfull, npart):
    m_full = sums_ref[0:1, :] / nfull
    m_part = sums_ref[1:2, :] / npart
    blk = x_ref[...]
    xb_ref[...] = jnp.sign(blk - m_full).astype(xb_ref.dtype)
    xtb_ref[...] = jnp.sign(blk - m_part).astype(xtb_ref.dtype)


_NH1 = _ND0 // _HBLK
_NH2 = _ND1 // _FBLK
_HKEEP = 4400


def _layer0_body(aggA_ref, aggB_ref, cnt_ref, xtb_ref, wlA_ref, wlB_ref,
                 wr_ref, b_ref, hb_ref, htb_ref, h_sc, hsums_sc):
    i = pl.program_id(0)

    @pl.when(i == 0)
    def _():
        hsums_sc[...] = jnp.zeros_like(hsums_sc)

    @pl.when(i < _NH1)
    def _():
        cnt = jnp.maximum(cnt_ref[...], 1.0)
        aggA = aggA_ref[...].astype(jnp.float32)
        aggB = aggB_ref[...].astype(jnp.float32)
        hblk = (jnp.dot(aggA / cnt, wlA_ref[...],
                        preferred_element_type=jnp.float32)
                + jnp.dot(aggB / cnt, wlB_ref[...],
                          preferred_element_type=jnp.float32)
                + jnp.dot(xtb_ref[...].astype(jnp.float32), wr_ref[...],
                          preferred_element_type=jnp.float32)
                + b_ref[...])
        hblk = jnp.maximum(hblk, 0.0)

        @pl.when(i < _HKEEP // _HBLK)
        def _():
            h_sc[pl.ds(i * _HBLK, _HBLK), :] = hblk

        hsums_sc[0:1, :] += jnp.sum(hblk, axis=0, keepdims=True)
        rows = i * _HBLK + lax.broadcasted_iota(jnp.int32, (_HBLK, 1), 0)
        hsums_sc[1:2, :] += jnp.sum(jnp.where(rows < _ND1, hblk, 0.0),
                                    axis=0, keepdims=True)

    @pl.when(i >= _NH1)
    def _():
        j = i - _NH1
        m_full = hsums_sc[0:1, :] / float(_ND0)
        m_part = hsums_sc[1:2, :] / float(_ND1)
        blk = h_sc[pl.ds(j * _FBLK, _FBLK), :]
        hb_ref[...] = jnp.sign(blk - m_full).astype(hb_ref.dtype)
        htb_ref[...] = jnp.sign(blk - m_part).astype(htb_ref.dtype)


def _final_body(aggA_ref, aggB_ref, cntA_ref, cntB_ref, htb_ref,
                wl_ref, wr_ref, b_ref, out_ref):
    cnt = jnp.maximum(cntA_ref[...] + cntB_ref[...], 1.0)
    ma = (aggA_ref[...].astype(jnp.float32)
          + aggB_ref[...].astype(jnp.float32)) / cnt
    z = (jnp.dot(ma, wl_ref[...], preferred_element_type=jnp.float32)
         + jnp.dot(htb_ref[...].astype(jnp.float32), wr_ref[...],
                   preferred_element_type=jnp.float32)
         + b_ref[...])
    z = z - jnp.max(z, axis=1, keepdims=True)
    z = z - jnp.log(jnp.sum(jnp.exp(z), axis=1, keepdims=True))
    out_ref[...] = z


_IB = 1024
_CPB = _IB // _CH
_CAP0 = _EPT0 + 2 * _CH


def _make_agg_body(ept, nch, half, garb, z_stripe, out_rows, out_tiles,
                   core_splits_edges, col_split=False):
    nb = ept // _IB
    assert nb % 2 == 0 or nb == 1
    assert nch == nb * _CPB
    assert _CPB >= _NRING
    cw = (max(z_stripe, out_rows) + 15) // 16 * 16

    def body(tab, src, dst, zr, agg_out, cnt_out, *sc):
        srcb = [sc[0], sc[1]]
        dstb = [sc[2], sc[3]]
        rows = list(sc[4:4 + _NRING])
        sidx = list(sc[4 + _NRING:4 + 2 * _NRING])
        gidx = list(sc[4 + 2 * _NRING:4 + 3 * _NRING])
        ones_v, cstage_v, agg_sh, cnt_sh = sc[4 + 3 * _NRING:8 + 3 * _NRING]
        gsem = list(sc[8 + 3 * _NRING:8 + 4 * _NRING])
        isem = list(sc[8 + 4 * _NRING:10 + 4 * _NRING])
        c = lax.axis_index("c")
        s = lax.axis_index("s")
        dbase = 0 if (core_splits_edges or col_split) else c * half
        ebase = (c * (ept * _NS) if core_splits_edges else 0) + s * ept

        def zbody(i, carry):
            cstage_v[pl.ds(i * 16, 16)] = jnp.zeros((16,), jnp.float32)
            return carry

        lax.fori_loop(0, cw // 16, zbody, 0)
        pltpu.sync_copy(zr.at[pl.ds(s * z_stripe, z_stripe)],
                        agg_sh.at[pl.ds(s * z_stripe, z_stripe)])
        pltpu.sync_copy(cstage_v.at[pl.ds(0, z_stripe)],
                        cnt_sh.at[pl.ds(s * z_stripe, z_stripe)])
        for k in range(_CH // 16):
            ones_v[pl.ds(k * 16, 16)] = jnp.full((16,), 1.0, jnp.float32)
        pltpu.sync_copy(src.at[pl.ds(ebase, _IB)], srcb[0])
        pltpu.sync_copy(dst.at[pl.ds(ebase, _IB)], dstb[0])
        if nb > 1:
            pltpu.async_copy(src.at[pl.ds(ebase + _IB, _IB)], srcb[1], isem[1])
            pltpu.async_copy(dst.at[pl.ds(ebase + _IB, _IB)], dstb[1], isem[1])
        plsc.subcore_barrier()

        garb_u = jnp.uint32(garb)

        def compute_sidx(dbuf, off, xbuf):
            for k in range(_CH // 16):
                d = dbuf[pl.ds(off + k * 16, 16)]
                loc = plsc.bitcast(d - dbase, jnp.uint32)
                xbuf[pl.ds(k * 16, 16)] = plsc.bitcast(
                    jnp.minimum(loc, garb_u), jnp.int32)

        def gather(sbuf, off, b):
            if not col_split:
                return pltpu.async_copy(
                    tab.at[sbuf.at[pl.ds(off, _CH)]], rows[b], gsem[b])
            for k in range(_CH // 16):
                sv = sbuf[pl.ds(off + k * 16, 16)]
                gidx[b][pl.ds(k * 16, 16)] = (sv << 1) + c
            return pltpu.async_copy(tab.at[gidx[b]], rows[b], gsem[b])

        def gather_wait(sbuf, off, b):
            if not col_split:
                pltpu.make_async_copy(
                    tab.at[sbuf.at[pl.ds(off, _CH)]], rows[b], gsem[b]).wait()
            else:
                pltpu.make_async_copy(tab.at[gidx[b]], rows[b], gsem[b]).wait()

        def scatter(b):
            pltpu.sync_copy(rows[b], agg_sh.at[sidx[b]], add=True)
            if col_split:
                @pl.when(c == 0)
                def _():
                    pltpu.sync_copy(ones_v, cnt_sh.at[sidx[b]], add=True)
            else:
                pltpu.sync_copy(ones_v, cnt_sh.at[sidx[b]], add=True)

        for b in range(_NRING):
            compute_sidx(dstb[0], b * _CH, sidx[b])
            gather(srcb[0], b * _CH, b)

        def pair_body(t, carry):
            for hm in range(2):
                m = 2 * t + hm
                bufm = hm
                bufn = 1 - hm
                for p in range(_CPB):
                    b = p % _NRING
                    gather_wait(srcb[bufm], p * _CH, b)
                    scatter(b)
                    if p == _CPB - _NRING:
                        @pl.when(m + 2 < nb)
                        def _():
                            off = ebase + (m + 2) * _IB
                            pltpu.async_copy(src.at[pl.ds(off, _IB)],
                                             srcb[bufm], isem[bufm])
                            pltpu.async_copy(dst.at[pl.ds(off, _IB)],
                                             dstb[bufm], isem[bufm])

                        @pl.when(m + 1 < nb)
                        def _():
                            off = ebase + (m + 1) * _IB
                            pltpu.make_async_copy(
                                src.at[pl.ds(off, _IB)], srcb[bufn],
                                isem[bufn]).wait()
                            pltpu.make_async_copy(
                                dst.at[pl.ds(off, _IB)], dstb[bufn],
                                isem[bufn]).wait()
                    j = m * _CPB + p + _NRING
                    if p < _CPB - _NRING:
                        jbuf, joff = bufm, (p + _NRING) * _CH
                    else:
                        jbuf, joff = bufn, (p + _NRING - _CPB) * _CH

                    @pl.when(j < nch)
                    def _():
                        compute_sidx(dstb[jbuf], joff, sidx[b])
                        gather(srcb[jbuf], joff, b)
            return carry

        lax.fori_loop(0, max(nb // 2, 1), pair_body, 0)
        plsc.subcore_barrier()

        @pl.when(s < out_tiles)
        def _():
            pltpu.sync_copy(agg_sh.at[pl.ds(s * out_rows, out_rows)],
                            agg_out.at[pl.ds(c * half + s * out_rows, out_rows)])

        cnt_base = (s * out_rows) if col_split else (c * half + s * out_rows)
        cnt_write = ((c == 0) & (s < out_tiles)) if col_split else (s < out_tiles)

        @pl.when(cnt_write)
        def _():
            pltpu.sync_copy(cnt_sh.at[pl.ds(s * out_rows, out_rows)],
                            cstage_v.at[pl.ds(0, out_rows)])
            pltpu.sync_copy(cstage_v.at[pl.ds(0, out_rows)],
                            cnt_out.at[pl.ds(cnt_base, out_rows)])

    return body


def _agg_scratch(rows, cstage, w, dtype=jnp.float32):
    return ([pltpu.VMEM((_IB,), jnp.int32)] * 2
            + [pltpu.VMEM((_IB,), jnp.int32)] * 2
            + [pltpu.VMEM((_CH, w), dtype)] * _NRING
            + [pltpu.VMEM((_CH,), jnp.int32)] * _NRING
            + [pltpu.VMEM((_CH,), jnp.int32)] * _NRING
            + [pltpu.VMEM((_CH,), jnp.float32),
               pltpu.VMEM((cstage,), jnp.float32),
               pltpu.VMEM_SHARED((rows, w), dtype),
               pltpu.VMEM_SHARED((rows,), jnp.float32)]
            + [pltpu.SemaphoreType.DMA] * (_NRING + 2))


_ROWS0C = 20480


@functools.cache
def _sc_kernels():
    mesh = plsc.VectorSubcoreMesh(core_axis_name="c", subcore_axis_name="s",
                                  num_cores=_NC, num_subcores=_NS)
    agg0 = pl.kernel(
        _make_agg_body(_EPT0, _NCH0, _ND0, _ROWS0C - 1, z_stripe=1280,
                       out_rows=2000, out_tiles=10, core_splits_edges=False,
                       col_split=True),
        out_type=[jax.ShapeDtypeStruct((_NC * _ND0, _IN_CH // 2), jnp.bfloat16),
                  jax.ShapeDtypeStruct((_ND0,), jnp.float32)],
        mesh=mesh,
        scratch_types=_agg_scratch(_ROWS0C, 2000, _IN_CH // 2, jnp.bfloat16),
        compiler_params=pltpu.CompilerParams(use_tc_tiling_on_sc=False),
    )
    agg1 = pl.kernel(
        _make_agg_body(_EPT1, _NCH1, _ND1, _ND1 - 1, z_stripe=256,
                       out_rows=256, out_tiles=_NS, core_splits_edges=True),
        out_type=[jax.ShapeDtypeStruct((_NC * _ND1, _HID), jnp.bfloat16),
                  jax.ShapeDtypeStruct((_NC * _ND1,), jnp.float32)],
        mesh=mesh,
        scratch_types=_agg_scratch(_ND1, 256, _HID, jnp.bfloat16),
        compiler_params=pltpu.CompilerParams(use_tc_tiling_on_sc=False),
    )
    return agg0, agg1


def kernel(x, edge_index0, edge_index1, size0_dst, size1_dst,
           W_l0, W_r0, b0, W_l1, W_r1, b1):
    f32 = jnp.float32
    x = x.astype(f32)
    src0 = edge_index0[0]
    dst0 = edge_index0[1]
    src1 = edge_index1[0]
    dst1 = edge_index1[1]

    sums = pl.pallas_call(
        _colsum_body,
        grid=(_N_SRC // _CBLK,),
        in_specs=[pl.BlockSpec((_CBLK, _IN_CH), lambda i: (i, 0))],
        out_specs=pl.BlockSpec((8, _IN_CH), lambda i: (0, 0)),
        out_shape=jax.ShapeDtypeStruct((8, _IN_CH), f32),
    )(x)

    xb, xtb = pl.pallas_call(
        functools.partial(_binarize_body, nfull=float(_N_SRC), npart=float(_ND0)),
        grid=(_ND0 // _BBLK,),
        in_specs=[pl.BlockSpec((8, _IN_CH), lambda i: (0, 0)),
                  pl.BlockSpec((_BBLK, _IN_CH), lambda i: (i, 0))],
        out_specs=[pl.BlockSpec((_BBLK, _IN_CH), lambda i: (i, 0))] * 2,
        out_shape=[jax.ShapeDtypeStruct((_ND0, _IN_CH), jnp.bfloat16)] * 2,
    )(sums, x)

    zr0 = jnp.zeros((_ROWS0C, _IN_CH // 2), jnp.bfloat16)
    zr1 = jnp.zeros((_NS * 256, _HID), jnp.bfloat16)
    npad = _E0P - _E0
    src0p = jnp.concatenate([src0, jnp.zeros((npad,), jnp.int32)])
    dst0p = jnp.concatenate([dst0, jnp.full((npad,), _ND0, jnp.int32)])
    sc_agg0, sc_agg1 = _sc_kernels()
    agg0p, cnt0 = sc_agg0(xb.reshape(_NC * _ND0, _IN_CH // 2), src0p, dst0p,
                          zr0)

    p1 = lambda i: (jnp.minimum(i, _NH1 - 1), 0)
    p2 = lambda i: (jnp.maximum(i - _NH1, 0), 0)
    hb, htb = pl.pallas_call(
        _layer0_body,
        grid=(_NH1 + _NH2,),
        in_specs=[pl.BlockSpec((_HBLK, _IN_CH // 2), p1),
                  pl.BlockSpec((_HBLK, _IN_CH // 2), p1),
                  pl.BlockSpec((_HBLK, 1), p1),
                  pl.BlockSpec((_HBLK, _IN_CH), p1),
                  pl.BlockSpec((_IN_CH // 2, _HID), lambda i: (0, 0)),
                  pl.BlockSpec((_IN_CH // 2, _HID), lambda i: (0, 0)),
                  pl.BlockSpec((_IN_CH, _HID), lambda i: (0, 0)),
                  pl.BlockSpec((1, _HID), lambda i: (0, 0))],
        out_specs=[pl.BlockSpec((_FBLK, _HID), p2)] * 2,
        out_shape=[jax.ShapeDtypeStruct((_ND1, _HID), jnp.bfloat16)] * 2,
        scratch_shapes=[pltpu.VMEM((_HKEEP, _HID), f32),
                        pltpu.VMEM((8, _HID), f32)],
    )(agg0p[:_ND0], agg0p[_ND0:], cnt0.reshape(_ND0, 1), xtb,
      W_l0[:_IN_CH // 2], W_l0[_IN_CH // 2:], W_r0, b0.reshape(1, _HID))

    agg1p, cnt1p = sc_agg1(hb, src1, dst1, zr1)

    out = pl.pallas_call(
        _final_body,
        grid=(_ND1 // _FBLK,),
        in_specs=[pl.BlockSpec((_FBLK, _HID), lambda i: (i, 0)),
                  pl.BlockSpec((_FBLK, _HID), lambda i: (i, 0)),
                  pl.BlockSpec((_FBLK, 1), lambda i: (i, 0)),
                  pl.BlockSpec((_FBLK, 1), lambda i: (i, 0)),
                  pl.BlockSpec((_FBLK, _HID), lambda i: (i, 0)),
                  pl.BlockSpec((_HID, _OUT), lambda i: (0, 0)),
                  pl.BlockSpec((_HID, _OUT), lambda i: (0, 0)),
                  pl.BlockSpec((1, _OUT), lambda i: (0, 0))],
        out_specs=pl.BlockSpec((_FBLK, _OUT), lambda i: (i, 0)),
        out_shape=jax.ShapeDtypeStruct((_ND1, _OUT), f32),
    )(agg1p[:_ND1], agg1p[_ND1:], cnt1p[:_ND1].reshape(_ND1, 1),
      cnt1p[_ND1:].reshape(_ND1, 1), htb, W_l1, W_r1, b1.reshape(1, _OUT))
    return out
